# baseline (speedup 1.0000x reference)
import functools

import jax
import jax.numpy as jnp
from jax import lax
from jax.experimental import pallas as pl
from jax.experimental.pallas import tpu as pltpu
from jax.experimental.pallas import tpu_sc as plsc

N = 10000
H = 128
E = 160000
STEPS = 4

NC = 2
NS = 16
CH = 128
CPT = 79
TPT = CPT * CH
EP = NS * TPT
NP = 10240
DUMMY = N

BN = 512
BE = 1024

NW = NC * NS
CPG = 40
TPW = CPG * CH
EPW = NW * TPW

_mesh = plsc.VectorSubcoreMesh(core_axis_name="c", subcore_axis_name="s")


NBUF = 3


@functools.partial(
    pl.kernel,
    out_type=jax.ShapeDtypeStruct((EPW, H), jnp.float32),
    mesh=_mesh,
    scratch_types=[
        pltpu.VMEM((CPG, CH), jnp.int32),
        pltpu.VMEM((CPG, CH), jnp.int32),
        pltpu.VMEM((NBUF, CH, H), jnp.float32),
        pltpu.VMEM((NBUF, CH, H), jnp.float32),
        pltpu.SemaphoreType.DMA,
        pltpu.SemaphoreType.DMA,
        pltpu.SemaphoreType.DMA,
    ],
)
def _sc_gather(ab_hbm, gidx_hbm, out_hbm, idxA, idxB, bufA, bufB,
               semA, semB, semO):
    c = lax.axis_index("c")
    s = lax.axis_index("s")
    w = s * NC + c
    base = w * TPW

    pltpu.sync_copy(gidx_hbm.at[0, w], idxA)
    pltpu.sync_copy(gidx_hbm.at[1, w], idxB)

    def start_gather(j, slot):
        pltpu.async_copy(ab_hbm.at[idxA.at[j]], bufA.at[slot], semA)
        pltpu.async_copy(ab_hbm.at[idxB.at[j]], bufB.at[slot], semB)

    def wait_gather(slot):
        pltpu.make_async_copy(ab_hbm.at[pl.ds(0, CH)], bufA.at[slot], semA).wait()
        pltpu.make_async_copy(ab_hbm.at[pl.ds(0, CH)], bufB.at[slot], semB).wait()

    def wait_out(slot=0):
        pltpu.make_async_copy(ab_hbm.at[pl.ds(0, CH)], bufA.at[slot], semO).wait()

    for j in range(NBUF - 1):
        start_gather(j, j)

    def body(j, _):
        slot = lax.rem(j, NBUF)
        wait_gather(slot)

        @pl.when(j + NBUF - 1 < CPG)
        def _():
            @pl.when(j >= 1)
            def _():
                wait_out()
            start_gather(j + NBUF - 1, lax.rem(j + NBUF - 1, NBUF))

        @plsc.parallel_loop(0, CH, 1, unroll=4)
        def _add_row(r):
            for k in range(H // 16):
                sl = (slot, r, pl.ds(k * 16, 16))
                bufA[sl] = bufA[sl] + bufB[sl]
        pltpu.async_copy(bufA.at[slot],
                         out_hbm.at[pl.ds(base + j * CH, CH)], semO)
        return 0

    lax.fori_loop(0, CPG, body, 0)
    for _ in range(min(NBUF, CPG)):
        wait_out()


@functools.partial(
    pl.kernel,
    out_type=jax.ShapeDtypeStruct((2, NP, H), jnp.float32),
    mesh=_mesh,
    scratch_types=[
        pltpu.VMEM((CPG, CH), jnp.int32),
        pltpu.VMEM((2, CH, H), jnp.float32),
        pltpu.VMEM_SHARED((NP, H), jnp.float32),
        pltpu.SemaphoreType.DMA,
    ],
)
def _sc_scatter(m_hbm, sidx_hbm, out_hbm, idxD, bufM, agg, semM):
    c = lax.axis_index("c")
    s = lax.axis_index("s")
    w = s * NC + c
    base = w * TPW
    stripe = NP // NS

    pltpu.sync_copy(sidx_hbm.at[w], idxD)

    def zero_row(r, _):
        for k in range(H // 16):
            bufM[0, r, pl.ds(k * 16, 16)] = jnp.zeros((16,), jnp.float32)
        return 0

    lax.fori_loop(0, CH, zero_row, 0)
    for t in range(stripe // CH):
        pltpu.sync_copy(bufM.at[0], agg.at[pl.ds(s * stripe + t * CH, CH)])
    plsc.subcore_barrier()

    def start_load(j, slot):
        pltpu.async_copy(m_hbm.at[pl.ds(base + j * CH, CH)],
                         bufM.at[slot], semM)

    def wait_load(slot):
        pltpu.make_async_copy(m_hbm.at[pl.ds(0, CH)], bufM.at[slot], semM).wait()

    start_load(0, 0)

    def body(j, _):
        slot = lax.rem(j, 2)
        wait_load(slot)

        @pl.when(j + 1 < CPG)
        def _():
            start_load(j + 1, 1 - slot)

        pltpu.sync_copy(bufM.at[slot], agg.at[idxD.at[j]], add=True)
        return 0

    lax.fori_loop(0, CPG, body, 0)
    plsc.subcore_barrier()
    for t in range(stripe // CH):
        sl = pl.ds(s * stripe + t * CH, CH)
        pltpu.sync_copy(agg.at[sl], out_hbm.at[c, sl])


CSTR = NP * 8 // NS


@functools.partial(
    pl.kernel,
    out_type=jax.ShapeDtypeStruct((2, NP * 8), jnp.float32),
    mesh=_mesh,
    scratch_types=[
        pltpu.VMEM((CPT, CH), jnp.int32),
        pltpu.VMEM((CPT, CH), jnp.int32),
        pltpu.VMEM((2, CH), jnp.int32),
        pltpu.VMEM((1, CH), jnp.int32),
        pltpu.VMEM((CH,), jnp.float32),
        pltpu.VMEM((CSTR,), jnp.float32),
        pltpu.VMEM_SHARED((NP * 8,), jnp.float32),
        pltpu.SemaphoreType.DMA,
    ],
)
def _sc_count(q_hbm, didx_hbm, ridx_hbm, out_hbm,
              idxD, idxS, qbuf, ibuf, ones, zstage, cnt, semQ):
    c = lax.axis_index("c")
    s = lax.axis_index("s")

    pltpu.sync_copy(didx_hbm.at[c, s], idxD)
    pltpu.sync_copy(ridx_hbm.at[c, s], idxS)

    def zrow(r, _):
        zstage[pl.ds(r * 16, 16)] = jnp.zeros((16,), jnp.float32)
        return 0

    lax.fori_loop(0, CSTR // 16, zrow, 0)
    for r in range(CH // 16):
        ones[pl.ds(r * 16, 16)] = jnp.full((16,), 1.0, jnp.float32)
    pltpu.sync_copy(zstage, cnt.at[pl.ds(s * CSTR, CSTR)])
    plsc.subcore_barrier()

    def start_q(j, slot):
        pltpu.async_copy(q_hbm.at[idxS.at[j]], qbuf.at[slot], semQ)

    def wait_q(slot):
        pltpu.make_async_copy(q_hbm.at[pl.ds(0, CH)], qbuf.at[slot], semQ).wait()

    start_q(0, 0)

    def body(j, _):
        slot = lax.rem(j, 2)
        wait_q(slot)

        @pl.when(j + 1 < CPT)
        def _():
            start_q(j + 1, 1 - slot)

        for k in range(CH // 16):
            sl = pl.ds(k * 16, 16)
            ibuf[0, sl] = idxD[j, sl] * 8 + qbuf[slot, sl]
        pltpu.sync_copy(ones, cnt.at[ibuf.at[0]], add=True)
        return 0

    lax.fori_loop(0, CPT, body, 0)
    plsc.subcore_barrier()
    pltpu.sync_copy(cnt.at[pl.ds(s * CSTR, CSTR)],
                    out_hbm.at[c, pl.ds(s * CSTR, CSTR)])


def _mlp_body(g_ref, w2_ref, b2_ref, w3_ref, b3_ref, w4_ref, b4_ref, out_ref):
    z = jnp.maximum(g_ref[...], 0.0).astype(jnp.bfloat16)
    z = jnp.maximum(jnp.dot(z, w2_ref[...], preferred_element_type=jnp.float32)
                    + b2_ref[...], 0.0).astype(jnp.bfloat16)
    z = jnp.maximum(jnp.dot(z, w3_ref[...], preferred_element_type=jnp.float32)
                    + b3_ref[...], 0.0).astype(jnp.bfloat16)
    out_ref[...] = (jnp.dot(z, w4_ref[...], preferred_element_type=jnp.float32)
                    + b4_ref[...])


_mlp_call = pl.pallas_call(
    _mlp_body,
    grid=(EPW // BE,),
    in_specs=[
        pl.BlockSpec((BE, H), lambda i: (i, 0)),
        pl.BlockSpec((H, H), lambda i: (0, 0)),
        pl.BlockSpec((1, H), lambda i: (0, 0)),
        pl.BlockSpec((H, H), lambda i: (0, 0)),
        pl.BlockSpec((1, H), lambda i: (0, 0)),
        pl.BlockSpec((H, H), lambda i: (0, 0)),
        pl.BlockSpec((1, H), lambda i: (0, 0)),
    ],
    out_specs=pl.BlockSpec((BE, H), lambda i: (i, 0)),
    out_shape=jax.ShapeDtypeStruct((EPW, H), jnp.float32),
)


def _pair_mlp_body(emb_ref, w1_ref, b1_ref, w2_ref, b2_ref, w3_ref, b3_ref,
                   w4_ref, b4_ref, out_ref):
    e = emb_ref[...]
    rows = [jnp.concatenate([e[a], e[b]]) for a in range(3) for b in range(3)]
    z0 = jnp.concatenate([jnp.stack(rows),
                          jnp.zeros((7, 2 * H), jnp.float32)]
                         ).astype(jnp.bfloat16)
    for t in range(2):
        z = jnp.maximum(jnp.dot(z0, w1_ref[t], preferred_element_type=jnp.float32)
                        + b1_ref[t], 0.0).astype(jnp.bfloat16)
        z = jnp.maximum(jnp.dot(z, w2_ref[t], preferred_element_type=jnp.float32)
                        + b2_ref[t], 0.0).astype(jnp.bfloat16)
        z = jnp.maximum(jnp.dot(z, w3_ref[t], preferred_element_type=jnp.float32)
                        + b3_ref[t], 0.0).astype(jnp.bfloat16)
        out_ref[t] = (jnp.dot(z, w4_ref[t], preferred_element_type=jnp.float32)
                      + b4_ref[t])


_pair_mlp_call = pl.pallas_call(
    _pair_mlp_body,
    out_shape=jax.ShapeDtypeStruct((2, 16, H), jnp.float32),
)


def _combine_body(cnt_ref, q_ref, m9_ref, out_ref):
    q = q_ref[0, 0]
    qc = q[:, None]
    m9 = m9_ref[...]
    for t in range(2):
        cnt = cnt_ref[t]
        acc = None
        for c in range(3):
            row = jnp.where(qc == 0, m9[t, c * 3 + 0][None, :],
                            jnp.where(qc == 1, m9[t, c * 3 + 1][None, :],
                                      m9[t, c * 3 + 2][None, :]))
            term = cnt[:, c][:, None] * row
            acc = term if acc is None else acc + term
        out_ref[t] = acc


_combine_call = pl.pallas_call(
    _combine_body,
    grid=(NP // BN,),
    in_specs=[
        pl.BlockSpec((2, BN, 8), lambda i: (0, i, 0)),
        pl.BlockSpec((1, 1, BN), lambda i: (i, 0, 0)),
        pl.BlockSpec((2, 16, H), lambda i: (0, 0, 0)),
    ],
    out_specs=pl.BlockSpec((2, BN, H), lambda i: (0, i, 0)),
    out_shape=jax.ShapeDtypeStruct((2, NP, H), jnp.float32),
)


def _init_body(q_ref, emb_ref, wp_ref, bp_ref, x_ref, ab_ref):
    q = q_ref[0, 0]
    e = emb_ref[...]
    qc = q[:, None]
    x = jnp.where(qc == 0, e[0][None, :],
                  jnp.where(qc == 1, e[1][None, :], e[2][None, :]))
    x_ref[...] = x
    xb = x.astype(jnp.bfloat16)
    for t in range(4):
        ab_ref[t] = (jnp.dot(xb, wp_ref[t], preferred_element_type=jnp.float32)
                     + bp_ref[t])


_init_call = pl.pallas_call(
    _init_body,
    grid=(NP // BN,),
    in_specs=[
        pl.BlockSpec((1, 1, BN), lambda i: (i, 0, 0)),
        pl.BlockSpec((3, H), lambda i: (0, 0)),
        pl.BlockSpec((4, H, H), lambda i: (0, 0, 0)),
        pl.BlockSpec((4, 1, H), lambda i: (0, 0, 0)),
    ],
    out_specs=[
        pl.BlockSpec((BN, H), lambda i: (i, 0)),
        pl.BlockSpec((4, BN, H), lambda i: (0, i, 0)),
    ],
    out_shape=[
        jax.ShapeDtypeStruct((NP, H), jnp.float32),
        jax.ShapeDtypeStruct((4, NP, H), jnp.float32),
    ],
)


def _lstm_math(x, mI0, mI1, mE0, mE1, h, c, wih, whh):
    mI = mI0 + mI1
    mE = mE0 + mE1
    gates = (jnp.dot(x, wih[:H], preferred_element_type=jnp.float32)
             + jnp.dot(mI, wih[H:2 * H], preferred_element_type=jnp.float32)
             + jnp.dot(mE, wih[2 * H:], preferred_element_type=jnp.float32)
             + jnp.dot(h, whh, preferred_element_type=jnp.float32))
    i_g = gates[:, :H]
    f_g = gates[:, H:2 * H]
    g_g = gates[:, 2 * H:3 * H]
    o_g = gates[:, 3 * H:]
    c_new = jax.nn.sigmoid(f_g) * c + jax.nn.sigmoid(i_g) * jnp.tanh(g_g)
    h_new = jax.nn.sigmoid(o_g) * jnp.tanh(c_new)
    return h_new, c_new


def _lstm_proj_body(x_ref, aggI0_ref, aggI1_ref, aggE0_ref, aggE1_ref,
                    h_ref, c_ref, wih_ref, whh_ref,
                    wp_ref, bp_ref, h_out, c_out, ab_out):
    h_new, c_new = _lstm_math(x_ref[...], aggI0_ref[0], aggI1_ref[0],
                              aggE0_ref[0], aggE1_ref[0],
                              h_ref[...], c_ref[...], wih_ref[...], whh_ref[...])
    h_out[...] = h_new
    c_out[...] = c_new
    hb = h_new.astype(jnp.bfloat16)
    for t in range(4):
        ab_out[t] = (jnp.dot(hb, wp_ref[t], preferred_element_type=jnp.float32)
                     + bp_ref[t])


_lstm_proj_call = pl.pallas_call(
    _lstm_proj_body,
    grid=(NP // BN,),
    in_specs=[
        pl.BlockSpec((BN, H), lambda i: (i, 0)),
        pl.BlockSpec((1, BN, H), lambda i: (0, i, 0)),
        pl.BlockSpec((1, BN, H), lambda i: (1, i, 0)),
        pl.BlockSpec((1, BN, H), lambda i: (0, i, 0)),
        pl.BlockSpec((1, BN, H), lambda i: (1, i, 0)),
        pl.BlockSpec((BN, H), lambda i: (i, 0)),
        pl.BlockSpec((BN, H), lambda i: (i, 0)),
        pl.BlockSpec((3 * H, 4 * H), lambda i: (0, 0)),
        pl.BlockSpec((H, 4 * H), lambda i: (0, 0)),
        pl.BlockSpec((4, H, H), lambda i: (0, 0, 0)),
        pl.BlockSpec((4, 1, H), lambda i: (0, 0, 0)),
    ],
    out_specs=[
        pl.BlockSpec((BN, H), lambda i: (i, 0)),
        pl.BlockSpec((BN, H), lambda i: (i, 0)),
        pl.BlockSpec((4, BN, H), lambda i: (0, i, 0)),
    ],
    out_shape=[
        jax.ShapeDtypeStruct((NP, H), jnp.float32),
        jax.ShapeDtypeStruct((NP, H), jnp.float32),
        jax.ShapeDtypeStruct((4, NP, H), jnp.float32),
    ],
)


def _lstm_score_body(x_ref, aggI0_ref, aggI1_ref, aggE0_ref, aggE1_ref,
                     h_ref, c_ref, wih_ref, whh_ref, ws_ref, out_ref):
    h_new, _ = _lstm_math(x_ref[...], aggI0_ref[0], aggI1_ref[0],
                          aggE0_ref[0], aggE1_ref[0],
                          h_ref[...], c_ref[...], wih_ref[...], whh_ref[...])
    out_ref[...] = jnp.sum(h_new * ws_ref[0][None, :], axis=1)[None, :]


_lstm_score_call = pl.pallas_call(
    _lstm_score_body,
    grid=(NP // BN,),
    in_specs=[
        pl.BlockSpec((BN, H), lambda i: (i, 0)),
        pl.BlockSpec((1, BN, H), lambda i: (0, i, 0)),
        pl.BlockSpec((1, BN, H), lambda i: (1, i, 0)),
        pl.BlockSpec((1, BN, H), lambda i: (0, i, 0)),
        pl.BlockSpec((1, BN, H), lambda i: (1, i, 0)),
        pl.BlockSpec((BN, H), lambda i: (i, 0)),
        pl.BlockSpec((BN, H), lambda i: (i, 0)),
        pl.BlockSpec((3 * H, 4 * H), lambda i: (0, 0)),
        pl.BlockSpec((H, 4 * H), lambda i: (0, 0)),
        pl.BlockSpec((1, H), lambda i: (0, 0)),
    ],
    out_specs=pl.BlockSpec((1, BN), lambda i: (0, i)),
    out_shape=jax.ShapeDtypeStruct((1, NP), jnp.float32),
)


def _pad_to(v, length, fill):
    return jnp.concatenate(
        [v, jnp.full((length - v.shape[0],), fill, dtype=v.dtype)])


def kernel(cell_q, edge_intra, edge_inter, params):
    p = params

    wI, wE = p['intra_Ws'], p['inter_Ws']
    bI, bE = p['intra_bs'], p['inter_bs']
    wp = jnp.stack([wI[0][:H], wI[0][H:], wE[0][:H], wE[0][H:]]
                   ).astype(jnp.bfloat16)
    zb = jnp.zeros((H,), jnp.float32)
    bp = jnp.stack([zb, bI[0], zb, bE[0]]).reshape(4, 1, H)
    w2 = jnp.stack([wI[1], wE[1]]).astype(jnp.bfloat16)
    b2 = jnp.stack([bI[1], bE[1]]).reshape(2, 1, H)
    w3 = jnp.stack([wI[2], wE[2]]).astype(jnp.bfloat16)
    b3 = jnp.stack([bI[2], bE[2]]).reshape(2, 1, H)
    w4 = jnp.stack([wI[3], wE[3]]).astype(jnp.bfloat16)
    b4 = jnp.stack([bI[3], bE[3]]).reshape(2, 1, H)

    def gidx_type(edges, a_slab, b_slab):
        src = edges[0].astype(jnp.int32)
        dst = edges[1].astype(jnp.int32)
        ia = _pad_to(src + a_slab * NP, EPW, 0)
        ib = _pad_to(dst + b_slab * NP, EPW, 0)
        return jnp.stack([ia, ib]).reshape(2, NW, CPG, CH)

    gidxI = gidx_type(edge_intra, 0, 1)
    gidxE = gidx_type(edge_inter, 2, 3)
    sidxI = _pad_to(edge_intra[1].astype(jnp.int32), EPW,
                    DUMMY).reshape(NW, CPG, CH)
    sidxE = _pad_to(edge_inter[1].astype(jnp.int32), EPW,
                    DUMMY).reshape(NW, CPG, CH)

    cq32 = cell_q.astype(jnp.int32)
    ridx = jnp.stack([
        _pad_to(edge_intra[0].astype(jnp.int32), EP, 0),
        _pad_to(edge_inter[0].astype(jnp.int32), EP, 0),
    ]).reshape(2, NS, CPT, CH)
    didx = jnp.stack([
        _pad_to(edge_intra[1].astype(jnp.int32), EP, DUMMY),
        _pad_to(edge_inter[1].astype(jnp.int32), EP, DUMMY),
    ]).reshape(2, NS, CPT, CH)

    qpad = _pad_to(cq32, NP, 0)
    q3 = qpad.reshape(NP // BN, 1, BN)
    x, ab = _init_call(q3, p['digit_embed'], wp, bp)

    h = jnp.zeros((NP, H), jnp.float32)
    c = jnp.zeros((NP, H), jnp.float32)
    wih = p['W_ih']
    whh = p['W_hh']
    ws2d = p['w_score'].reshape(1, H)

    w1s = jnp.stack([wI[0], wE[0]]).astype(jnp.bfloat16)
    b1s = jnp.stack([bI[0], bE[0]]).reshape(2, 1, H)
    m9 = _pair_mlp_call(p['digit_embed'], w1s, b1s, w2, b2, w3, b3, w4, b4)
    cnt = _sc_count(qpad, didx, ridx).reshape(2, NP, 8)
    agg = _combine_call(cnt, q3, m9)
    znp = jnp.zeros((1, NP, H), jnp.float32)
    aggI = jnp.concatenate([agg[0:1], znp])
    aggE = jnp.concatenate([agg[1:2], znp])
    h, c, ab = _lstm_proj_call(x, aggI, aggI, aggE, aggE, h, c,
                               wih, whh, wp, bp)

    for step in range(1, STEPS):
        abflat = ab.reshape(4 * NP, H)
        gI = _sc_gather(abflat, gidxI)
        gE = _sc_gather(abflat, gidxE)
        mI = _mlp_call(gI, w2[0], b2[0], w3[0], b3[0], w4[0], b4[0])
        mE = _mlp_call(gE, w2[1], b2[1], w3[1], b3[1], w4[1], b4[1])
        aggI = _sc_scatter(mI, sidxI)
        aggE = _sc_scatter(mE, sidxE)
        if step < STEPS - 1:
            h, c, ab = _lstm_proj_call(x, aggI, aggI, aggE, aggE, h, c,
                                       wih, whh, wp, bp)
        else:
            logits = _lstm_score_call(x, aggI, aggI, aggE, aggE, h, c,
                                      wih, whh, ws2d)

    return logits[0, :N]


# speedup vs baseline: 1.2567x; 1.2567x over previous
import functools

import jax
import jax.numpy as jnp
from jax import lax
from jax.experimental import pallas as pl
from jax.experimental.pallas import tpu as pltpu
from jax.experimental.pallas import tpu_sc as plsc

N = 10000
H = 128
E = 160000
STEPS = 4

NC = 2
NS = 16
CH = 128
CPT = 79
TPT = CPT * CH
EP = NS * TPT
NP = 10240
DUMMY = N

BN = 512
BE = 1024

_mesh = plsc.VectorSubcoreMesh(core_axis_name="c", subcore_axis_name="s")


NBUF = 3


@functools.partial(
    pl.kernel,
    out_type=jax.ShapeDtypeStruct((2, EP, H), jnp.float32),
    mesh=_mesh,
    scratch_types=[
        pltpu.VMEM((CPT, CH), jnp.int32),
        pltpu.VMEM((CPT, CH), jnp.int32),
        pltpu.VMEM((NBUF, CH, H), jnp.float32),
        pltpu.VMEM((NBUF, CH, H), jnp.float32),
        pltpu.SemaphoreType.DMA,
        pltpu.SemaphoreType.DMA,
        pltpu.SemaphoreType.DMA,
    ],
)
def _sc_gather(ab_hbm, gidx_hbm, out_hbm, idxA, idxB, bufA, bufB,
               semA, semB, semO):
    c = lax.axis_index("c")
    s = lax.axis_index("s")
    base = s * TPT

    pltpu.sync_copy(gidx_hbm.at[c, 0, s], idxA)
    pltpu.sync_copy(gidx_hbm.at[c, 1, s], idxB)

    def start_gather(j, slot):
        pltpu.async_copy(ab_hbm.at[idxA.at[j]], bufA.at[slot], semA)
        pltpu.async_copy(ab_hbm.at[idxB.at[j]], bufB.at[slot], semB)

    def wait_gather(slot):
        pltpu.make_async_copy(ab_hbm.at[pl.ds(0, CH)], bufA.at[slot], semA).wait()
        pltpu.make_async_copy(ab_hbm.at[pl.ds(0, CH)], bufB.at[slot], semB).wait()

    def wait_out(slot=0):
        pltpu.make_async_copy(ab_hbm.at[pl.ds(0, CH)], bufA.at[slot], semO).wait()

    for j in range(NBUF - 1):
        start_gather(j, j)

    def body(j, _):
        slot = lax.rem(j, NBUF)
        wait_gather(slot)

        @pl.when(j + NBUF - 1 < CPT)
        def _():
            @pl.when(j >= 1)
            def _():
                wait_out()
            start_gather(j + NBUF - 1, lax.rem(j + NBUF - 1, NBUF))

        @plsc.parallel_loop(0, CH, 1, unroll=4)
        def _add_row(r):
            for k in range(H // 16):
                sl = (slot, r, pl.ds(k * 16, 16))
                bufA[sl] = bufA[sl] + bufB[sl]
        pltpu.async_copy(bufA.at[slot],
                         out_hbm.at[c, pl.ds(base + j * CH, CH)], semO)
        return 0

    lax.fori_loop(0, CPT, body, 0)
    for _ in range(min(NBUF, CPT)):
        wait_out()


@functools.partial(
    pl.kernel,
    out_type=jax.ShapeDtypeStruct((2, NP, H), jnp.float32),
    mesh=_mesh,
    scratch_types=[
        pltpu.VMEM((CPT, CH), jnp.int32),
        pltpu.VMEM((2, CH, H), jnp.float32),
        pltpu.VMEM_SHARED((NP, H), jnp.float32),
        pltpu.SemaphoreType.DMA,
    ],
)
def _sc_scatter(m_hbm, sidx_hbm, out_hbm, idxD, bufM, agg, semM):
    c = lax.axis_index("c")
    s = lax.axis_index("s")
    base = s * TPT
    stripe = NP // NS

    pltpu.sync_copy(sidx_hbm.at[c, s], idxD)

    def zero_row(r, _):
        for k in range(H // 16):
            bufM[0, r, pl.ds(k * 16, 16)] = jnp.zeros((16,), jnp.float32)
        return 0

    lax.fori_loop(0, CH, zero_row, 0)
    for t in range(stripe // CH):
        pltpu.sync_copy(bufM.at[0], agg.at[pl.ds(s * stripe + t * CH, CH)])
    plsc.subcore_barrier()

    def start_load(j, slot):
        pltpu.async_copy(m_hbm.at[c, pl.ds(base + j * CH, CH)],
                         bufM.at[slot], semM)

    def wait_load(slot):
        pltpu.make_async_copy(m_hbm.at[0, pl.ds(0, CH)], bufM.at[slot], semM).wait()

    start_load(0, 0)

    def body(j, _):
        slot = lax.rem(j, 2)
        wait_load(slot)

        @pl.when(j + 1 < CPT)
        def _():
            start_load(j + 1, 1 - slot)

        pltpu.sync_copy(bufM.at[slot], agg.at[idxD.at[j]], add=True)
        return 0

    lax.fori_loop(0, CPT, body, 0)
    plsc.subcore_barrier()
    for t in range(stripe // CH):
        sl = pl.ds(s * stripe + t * CH, CH)
        pltpu.sync_copy(agg.at[sl], out_hbm.at[c, sl])


CSTR = NP * 8 // NS


@functools.partial(
    pl.kernel,
    out_type=jax.ShapeDtypeStruct((2, NP * 8), jnp.float32),
    mesh=_mesh,
    scratch_types=[
        pltpu.VMEM((CPT, CH), jnp.int32),
        pltpu.VMEM((CPT, CH), jnp.int32),
        pltpu.VMEM((2, CH), jnp.int32),
        pltpu.VMEM((1, CH), jnp.int32),
        pltpu.VMEM((CH,), jnp.float32),
        pltpu.VMEM((CSTR,), jnp.float32),
        pltpu.VMEM_SHARED((NP * 8,), jnp.float32),
        pltpu.SemaphoreType.DMA,
    ],
)
def _sc_count(q_hbm, didx_hbm, ridx_hbm, out_hbm,
              idxD, idxS, qbuf, ibuf, ones, zstage, cnt, semQ):
    c = lax.axis_index("c")
    s = lax.axis_index("s")

    pltpu.sync_copy(didx_hbm.at[c, s], idxD)
    pltpu.sync_copy(ridx_hbm.at[c, s], idxS)

    def zrow(r, _):
        zstage[pl.ds(r * 16, 16)] = jnp.zeros((16,), jnp.float32)
        return 0

    lax.fori_loop(0, CSTR // 16, zrow, 0)
    for r in range(CH // 16):
        ones[pl.ds(r * 16, 16)] = jnp.full((16,), 1.0, jnp.float32)
    pltpu.sync_copy(zstage, cnt.at[pl.ds(s * CSTR, CSTR)])
    plsc.subcore_barrier()

    def start_q(j, slot):
        pltpu.async_copy(q_hbm.at[idxS.at[j]], qbuf.at[slot], semQ)

    def wait_q(slot):
        pltpu.make_async_copy(q_hbm.at[pl.ds(0, CH)], qbuf.at[slot], semQ).wait()

    start_q(0, 0)

    def body(j, _):
        slot = lax.rem(j, 2)
        wait_q(slot)

        @pl.when(j + 1 < CPT)
        def _():
            start_q(j + 1, 1 - slot)

        for k in range(CH // 16):
            sl = pl.ds(k * 16, 16)
            ibuf[0, sl] = idxD[j, sl] * 8 + qbuf[slot, sl]
        pltpu.sync_copy(ones, cnt.at[ibuf.at[0]], add=True)
        return 0

    lax.fori_loop(0, CPT, body, 0)
    plsc.subcore_barrier()
    pltpu.sync_copy(cnt.at[pl.ds(s * CSTR, CSTR)],
                    out_hbm.at[c, pl.ds(s * CSTR, CSTR)])


def _mlp_body(g_ref, w2_ref, b2_ref, w3_ref, b3_ref, w4_ref, b4_ref, out_ref):
    z = jnp.maximum(g_ref[0], 0.0).astype(jnp.bfloat16)
    z = jnp.maximum(jnp.dot(z, w2_ref[0], preferred_element_type=jnp.float32)
                    + b2_ref[0], 0.0).astype(jnp.bfloat16)
    z = jnp.maximum(jnp.dot(z, w3_ref[0], preferred_element_type=jnp.float32)
                    + b3_ref[0], 0.0).astype(jnp.bfloat16)
    out_ref[0] = (jnp.dot(z, w4_ref[0], preferred_element_type=jnp.float32)
                  + b4_ref[0])


_mlp_call = pl.pallas_call(
    _mlp_body,
    grid=(2, EP // BE),
    in_specs=[
        pl.BlockSpec((1, BE, H), lambda t, i: (t, i, 0)),
        pl.BlockSpec((1, H, H), lambda t, i: (t, 0, 0)),
        pl.BlockSpec((1, 1, H), lambda t, i: (t, 0, 0)),
        pl.BlockSpec((1, H, H), lambda t, i: (t, 0, 0)),
        pl.BlockSpec((1, 1, H), lambda t, i: (t, 0, 0)),
        pl.BlockSpec((1, H, H), lambda t, i: (t, 0, 0)),
        pl.BlockSpec((1, 1, H), lambda t, i: (t, 0, 0)),
    ],
    out_specs=pl.BlockSpec((1, BE, H), lambda t, i: (t, i, 0)),
    out_shape=jax.ShapeDtypeStruct((2, EP, H), jnp.float32),
)


def _pair_mlp_body(emb_ref, w1_ref, b1_ref, w2_ref, b2_ref, w3_ref, b3_ref,
                   w4_ref, b4_ref, out_ref):
    e = emb_ref[...]
    rows = [jnp.concatenate([e[a], e[b]]) for a in range(3) for b in range(3)]
    z0 = jnp.concatenate([jnp.stack(rows),
                          jnp.zeros((7, 2 * H), jnp.float32)]
                         ).astype(jnp.bfloat16)
    for t in range(2):
        z = jnp.maximum(jnp.dot(z0, w1_ref[t], preferred_element_type=jnp.float32)
                        + b1_ref[t], 0.0).astype(jnp.bfloat16)
        z = jnp.maximum(jnp.dot(z, w2_ref[t], preferred_element_type=jnp.float32)
                        + b2_ref[t], 0.0).astype(jnp.bfloat16)
        z = jnp.maximum(jnp.dot(z, w3_ref[t], preferred_element_type=jnp.float32)
                        + b3_ref[t], 0.0).astype(jnp.bfloat16)
        out_ref[t] = (jnp.dot(z, w4_ref[t], preferred_element_type=jnp.float32)
                      + b4_ref[t])


_pair_mlp_call = pl.pallas_call(
    _pair_mlp_body,
    out_shape=jax.ShapeDtypeStruct((2, 16, H), jnp.float32),
)


def _combine_body(cnt_ref, q_ref, m9_ref, out_ref):
    q = q_ref[0, 0]
    qc = q[:, None]
    m9 = m9_ref[...]
    for t in range(2):
        cnt = cnt_ref[t]
        acc = None
        for c in range(3):
            row = jnp.where(qc == 0, m9[t, c * 3 + 0][None, :],
                            jnp.where(qc == 1, m9[t, c * 3 + 1][None, :],
                                      m9[t, c * 3 + 2][None, :]))
            term = cnt[:, c][:, None] * row
            acc = term if acc is None else acc + term
        out_ref[t] = acc


_combine_call = pl.pallas_call(
    _combine_body,
    grid=(NP // BN,),
    in_specs=[
        pl.BlockSpec((2, BN, 8), lambda i: (0, i, 0)),
        pl.BlockSpec((1, 1, BN), lambda i: (i, 0, 0)),
        pl.BlockSpec((2, 16, H), lambda i: (0, 0, 0)),
    ],
    out_specs=pl.BlockSpec((2, BN, H), lambda i: (0, i, 0)),
    out_shape=jax.ShapeDtypeStruct((2, NP, H), jnp.float32),
)


def _init_body(q_ref, emb_ref, wp_ref, bp_ref, x_ref, ab_ref):
    q = q_ref[0, 0]
    e = emb_ref[...]
    qc = q[:, None]
    x = jnp.where(qc == 0, e[0][None, :],
                  jnp.where(qc == 1, e[1][None, :], e[2][None, :]))
    x_ref[...] = x
    xb = x.astype(jnp.bfloat16)
    for t in range(4):
        ab_ref[t] = (jnp.dot(xb, wp_ref[t], preferred_element_type=jnp.float32)
                     + bp_ref[t])


_init_call = pl.pallas_call(
    _init_body,
    grid=(NP // BN,),
    in_specs=[
        pl.BlockSpec((1, 1, BN), lambda i: (i, 0, 0)),
        pl.BlockSpec((3, H), lambda i: (0, 0)),
        pl.BlockSpec((4, H, H), lambda i: (0, 0, 0)),
        pl.BlockSpec((4, 1, H), lambda i: (0, 0, 0)),
    ],
    out_specs=[
        pl.BlockSpec((BN, H), lambda i: (i, 0)),
        pl.BlockSpec((4, BN, H), lambda i: (0, i, 0)),
    ],
    out_shape=[
        jax.ShapeDtypeStruct((NP, H), jnp.float32),
        jax.ShapeDtypeStruct((4, NP, H), jnp.float32),
    ],
)


def _lstm_math(x, mI, mE, h, c, wih, whh):
    gates = (jnp.dot(x, wih[:H], preferred_element_type=jnp.float32)
             + jnp.dot(mI, wih[H:2 * H], preferred_element_type=jnp.float32)
             + jnp.dot(mE, wih[2 * H:], preferred_element_type=jnp.float32)
             + jnp.dot(h, whh, preferred_element_type=jnp.float32))
    i_g = gates[:, :H]
    f_g = gates[:, H:2 * H]
    g_g = gates[:, 2 * H:3 * H]
    o_g = gates[:, 3 * H:]
    c_new = jax.nn.sigmoid(f_g) * c + jax.nn.sigmoid(i_g) * jnp.tanh(g_g)
    h_new = jax.nn.sigmoid(o_g) * jnp.tanh(c_new)
    return h_new, c_new


def _lstm_proj_body(x_ref, aggI_ref, aggE_ref, h_ref, c_ref, wih_ref, whh_ref,
                    wp_ref, bp_ref, h_out, c_out, ab_out):
    h_new, c_new = _lstm_math(x_ref[...], aggI_ref[0], aggE_ref[0],
                              h_ref[...], c_ref[...], wih_ref[...], whh_ref[...])
    h_out[...] = h_new
    c_out[...] = c_new
    hb = h_new.astype(jnp.bfloat16)
    for t in range(4):
        ab_out[t] = (jnp.dot(hb, wp_ref[t], preferred_element_type=jnp.float32)
                     + bp_ref[t])


_lstm_proj_call = pl.pallas_call(
    _lstm_proj_body,
    grid=(NP // BN,),
    in_specs=[
        pl.BlockSpec((BN, H), lambda i: (i, 0)),
        pl.BlockSpec((1, BN, H), lambda i: (0, i, 0)),
        pl.BlockSpec((1, BN, H), lambda i: (1, i, 0)),
        pl.BlockSpec((BN, H), lambda i: (i, 0)),
        pl.BlockSpec((BN, H), lambda i: (i, 0)),
        pl.BlockSpec((3 * H, 4 * H), lambda i: (0, 0)),
        pl.BlockSpec((H, 4 * H), lambda i: (0, 0)),
        pl.BlockSpec((4, H, H), lambda i: (0, 0, 0)),
        pl.BlockSpec((4, 1, H), lambda i: (0, 0, 0)),
    ],
    out_specs=[
        pl.BlockSpec((BN, H), lambda i: (i, 0)),
        pl.BlockSpec((BN, H), lambda i: (i, 0)),
        pl.BlockSpec((4, BN, H), lambda i: (0, i, 0)),
    ],
    out_shape=[
        jax.ShapeDtypeStruct((NP, H), jnp.float32),
        jax.ShapeDtypeStruct((NP, H), jnp.float32),
        jax.ShapeDtypeStruct((4, NP, H), jnp.float32),
    ],
)


def _lstm_score_body(x_ref, aggI_ref, aggE_ref, h_ref, c_ref, wih_ref,
                     whh_ref, ws_ref, out_ref):
    h_new, _ = _lstm_math(x_ref[...], aggI_ref[0], aggE_ref[0],
                          h_ref[...], c_ref[...], wih_ref[...], whh_ref[...])
    out_ref[...] = jnp.sum(h_new * ws_ref[0][None, :], axis=1)[None, :]


_lstm_score_call = pl.pallas_call(
    _lstm_score_body,
    grid=(NP // BN,),
    in_specs=[
        pl.BlockSpec((BN, H), lambda i: (i, 0)),
        pl.BlockSpec((1, BN, H), lambda i: (0, i, 0)),
        pl.BlockSpec((1, BN, H), lambda i: (1, i, 0)),
        pl.BlockSpec((BN, H), lambda i: (i, 0)),
        pl.BlockSpec((BN, H), lambda i: (i, 0)),
        pl.BlockSpec((3 * H, 4 * H), lambda i: (0, 0)),
        pl.BlockSpec((H, 4 * H), lambda i: (0, 0)),
        pl.BlockSpec((1, H), lambda i: (0, 0)),
    ],
    out_specs=pl.BlockSpec((1, BN), lambda i: (0, i)),
    out_shape=jax.ShapeDtypeStruct((1, NP), jnp.float32),
)


def _pad_to(v, length, fill):
    return jnp.concatenate(
        [v, jnp.full((length - v.shape[0],), fill, dtype=v.dtype)])


def kernel(cell_q, edge_intra, edge_inter, params):
    p = params

    wI, wE = p['intra_Ws'], p['inter_Ws']
    bI, bE = p['intra_bs'], p['inter_bs']
    wp = jnp.stack([wI[0][:H], wI[0][H:], wE[0][:H], wE[0][H:]]
                   ).astype(jnp.bfloat16)
    zb = jnp.zeros((H,), jnp.float32)
    bp = jnp.stack([zb, bI[0], zb, bE[0]]).reshape(4, 1, H)
    w2 = jnp.stack([wI[1], wE[1]]).astype(jnp.bfloat16)
    b2 = jnp.stack([bI[1], bE[1]]).reshape(2, 1, H)
    w3 = jnp.stack([wI[2], wE[2]]).astype(jnp.bfloat16)
    b3 = jnp.stack([bI[2], bE[2]]).reshape(2, 1, H)
    w4 = jnp.stack([wI[3], wE[3]]).astype(jnp.bfloat16)
    b4 = jnp.stack([bI[3], bE[3]]).reshape(2, 1, H)

    def gidx_type(edges, a_slab, b_slab):
        src = edges[0].astype(jnp.int32)
        dst = edges[1].astype(jnp.int32)
        ia = _pad_to(src + a_slab * NP, EP, 0)
        ib = _pad_to(dst + b_slab * NP, EP, 0)
        return jnp.stack([ia, ib]).reshape(2, NS, CPT, CH)

    gidx = jnp.stack([gidx_type(edge_intra, 0, 1),
                      gidx_type(edge_inter, 2, 3)])
    sidx = jnp.stack([
        _pad_to(edge_intra[1].astype(jnp.int32), EP, DUMMY),
        _pad_to(edge_inter[1].astype(jnp.int32), EP, DUMMY),
    ]).reshape(2, NS, CPT, CH)

    cq32 = cell_q.astype(jnp.int32)
    ridx = jnp.stack([
        _pad_to(edge_intra[0].astype(jnp.int32), EP, 0),
        _pad_to(edge_inter[0].astype(jnp.int32), EP, 0),
    ]).reshape(2, NS, CPT, CH)

    qpad = _pad_to(cq32, NP, 0)
    q3 = qpad.reshape(NP // BN, 1, BN)
    x, ab = _init_call(q3, p['digit_embed'], wp, bp)

    h = jnp.zeros((NP, H), jnp.float32)
    c = jnp.zeros((NP, H), jnp.float32)
    wih = p['W_ih']
    whh = p['W_hh']
    ws2d = p['w_score'].reshape(1, H)

    w1s = jnp.stack([wI[0], wE[0]]).astype(jnp.bfloat16)
    b1s = jnp.stack([bI[0], bE[0]]).reshape(2, 1, H)
    m9 = _pair_mlp_call(p['digit_embed'], w1s, b1s, w2, b2, w3, b3, w4, b4)
    cnt = _sc_count(qpad, sidx, ridx).reshape(2, NP, 8)
    agg = _combine_call(cnt, q3, m9)
    h, c, ab = _lstm_proj_call(x, agg, agg, h, c, wih, whh, wp, bp)

    for step in range(1, STEPS):
        g = _sc_gather(ab.reshape(4 * NP, H), gidx)
        m = _mlp_call(g, w2, b2, w3, b3, w4, b4)
        agg = _sc_scatter(m, sidx)
        if step < STEPS - 1:
            h, c, ab = _lstm_proj_call(x, agg, agg, h, c, wih, whh, wp, bp)
        else:
            logits = _lstm_score_call(x, agg, agg, h, c, wih, whh, ws2d)

    return logits[0, :N]


# speedup vs baseline: 1.2569x; 1.0002x over previous
import functools

import jax
import jax.numpy as jnp
from jax import lax
from jax.experimental import pallas as pl
from jax.experimental.pallas import tpu as pltpu
from jax.experimental.pallas import tpu_sc as plsc

N = 10000
H = 128
E = 160000
STEPS = 4

NC = 2
NS = 16
CH = 128
CPT = 79
TPT = CPT * CH
EP = NS * TPT
NP = 10240
DUMMY = N

BN = 512
BE = 1024

_mesh = plsc.VectorSubcoreMesh(core_axis_name="c", subcore_axis_name="s")


NBUF = 3


@functools.partial(
    pl.kernel,
    out_type=jax.ShapeDtypeStruct((2, EP, H), jnp.float32),
    mesh=_mesh,
    scratch_types=[
        pltpu.VMEM((CPT, CH), jnp.int32),
        pltpu.VMEM((CPT, CH), jnp.int32),
        pltpu.VMEM((NBUF, CH, H), jnp.float32),
        pltpu.VMEM((NBUF, CH, H), jnp.float32),
        pltpu.SemaphoreType.DMA,
        pltpu.SemaphoreType.DMA,
        pltpu.SemaphoreType.DMA,
    ],
)
def _sc_gather(ab_hbm, gidx_hbm, out_hbm, idxA, idxB, bufA, bufB,
               semA, semB, semO):
    c = lax.axis_index("c")
    s = lax.axis_index("s")
    base = s * TPT

    pltpu.sync_copy(gidx_hbm.at[c, 0, s], idxA)
    pltpu.sync_copy(gidx_hbm.at[c, 1, s], idxB)

    def start_gather(j, slot):
        pltpu.async_copy(ab_hbm.at[idxA.at[j]], bufA.at[slot], semA)
        pltpu.async_copy(ab_hbm.at[idxB.at[j]], bufB.at[slot], semB)

    def wait_gather(slot):
        pltpu.make_async_copy(ab_hbm.at[pl.ds(0, CH)], bufA.at[slot], semA).wait()
        pltpu.make_async_copy(ab_hbm.at[pl.ds(0, CH)], bufB.at[slot], semB).wait()

    def wait_out(slot=0):
        pltpu.make_async_copy(ab_hbm.at[pl.ds(0, CH)], bufA.at[slot], semO).wait()

    for j in range(NBUF - 1):
        start_gather(j, j)

    def body(j, _):
        slot = lax.rem(j, NBUF)
        wait_gather(slot)

        @pl.when(j + NBUF - 1 < CPT)
        def _():
            @pl.when(j >= 1)
            def _():
                wait_out()
            start_gather(j + NBUF - 1, lax.rem(j + NBUF - 1, NBUF))

        @plsc.parallel_loop(0, CH, 1, unroll=8)
        def _add_row(r):
            for k in range(H // 16):
                sl = (slot, r, pl.ds(k * 16, 16))
                bufA[sl] = bufA[sl] + bufB[sl]
        pltpu.async_copy(bufA.at[slot],
                         out_hbm.at[c, pl.ds(base + j * CH, CH)], semO)
        return 0

    lax.fori_loop(0, CPT, body, 0)
    for _ in range(min(NBUF, CPT)):
        wait_out()


@functools.partial(
    pl.kernel,
    out_type=jax.ShapeDtypeStruct((2, NP, H), jnp.float32),
    mesh=_mesh,
    scratch_types=[
        pltpu.VMEM((CPT, CH), jnp.int32),
        pltpu.VMEM((2, CH, H), jnp.float32),
        pltpu.VMEM_SHARED((NP, H), jnp.float32),
        pltpu.SemaphoreType.DMA,
    ],
)
def _sc_scatter(m_hbm, sidx_hbm, out_hbm, idxD, bufM, agg, semM):
    c = lax.axis_index("c")
    s = lax.axis_index("s")
    base = s * TPT
    stripe = NP // NS

    pltpu.sync_copy(sidx_hbm.at[c, s], idxD)

    def zero_row(r, _):
        for k in range(H // 16):
            bufM[0, r, pl.ds(k * 16, 16)] = jnp.zeros((16,), jnp.float32)
        return 0

    lax.fori_loop(0, CH, zero_row, 0)
    for t in range(stripe // CH):
        pltpu.sync_copy(bufM.at[0], agg.at[pl.ds(s * stripe + t * CH, CH)])
    plsc.subcore_barrier()

    def start_load(j, slot):
        pltpu.async_copy(m_hbm.at[c, pl.ds(base + j * CH, CH)],
                         bufM.at[slot], semM)

    def wait_load(slot):
        pltpu.make_async_copy(m_hbm.at[0, pl.ds(0, CH)], bufM.at[slot], semM).wait()

    start_load(0, 0)

    def body(j, _):
        slot = lax.rem(j, 2)
        wait_load(slot)

        @pl.when(j + 1 < CPT)
        def _():
            start_load(j + 1, 1 - slot)

        pltpu.sync_copy(bufM.at[slot], agg.at[idxD.at[j]], add=True)
        return 0

    lax.fori_loop(0, CPT, body, 0)
    plsc.subcore_barrier()
    for t in range(stripe // CH):
        sl = pl.ds(s * stripe + t * CH, CH)
        pltpu.sync_copy(agg.at[sl], out_hbm.at[c, sl])


CSTR = NP * 8 // NS


@functools.partial(
    pl.kernel,
    out_type=jax.ShapeDtypeStruct((2, NP * 8), jnp.float32),
    mesh=_mesh,
    scratch_types=[
        pltpu.VMEM((CPT, CH), jnp.int32),
        pltpu.VMEM((CPT, CH), jnp.int32),
        pltpu.VMEM((2, CH), jnp.int32),
        pltpu.VMEM((1, CH), jnp.int32),
        pltpu.VMEM((CH,), jnp.float32),
        pltpu.VMEM((CSTR,), jnp.float32),
        pltpu.VMEM_SHARED((NP * 8,), jnp.float32),
        pltpu.SemaphoreType.DMA,
    ],
)
def _sc_count(q_hbm, didx_hbm, ridx_hbm, out_hbm,
              idxD, idxS, qbuf, ibuf, ones, zstage, cnt, semQ):
    c = lax.axis_index("c")
    s = lax.axis_index("s")

    pltpu.sync_copy(didx_hbm.at[c, s], idxD)
    pltpu.sync_copy(ridx_hbm.at[c, s], idxS)

    def zrow(r, _):
        zstage[pl.ds(r * 16, 16)] = jnp.zeros((16,), jnp.float32)
        return 0

    lax.fori_loop(0, CSTR // 16, zrow, 0)
    for r in range(CH // 16):
        ones[pl.ds(r * 16, 16)] = jnp.full((16,), 1.0, jnp.float32)
    pltpu.sync_copy(zstage, cnt.at[pl.ds(s * CSTR, CSTR)])
    plsc.subcore_barrier()

    def start_q(j, slot):
        pltpu.async_copy(q_hbm.at[idxS.at[j]], qbuf.at[slot], semQ)

    def wait_q(slot):
        pltpu.make_async_copy(q_hbm.at[pl.ds(0, CH)], qbuf.at[slot], semQ).wait()

    start_q(0, 0)

    def body(j, _):
        slot = lax.rem(j, 2)
        wait_q(slot)

        @pl.when(j + 1 < CPT)
        def _():
            start_q(j + 1, 1 - slot)

        for k in range(CH // 16):
            sl = pl.ds(k * 16, 16)
            ibuf[0, sl] = idxD[j, sl] * 8 + qbuf[slot, sl]
        pltpu.sync_copy(ones, cnt.at[ibuf.at[0]], add=True)
        return 0

    lax.fori_loop(0, CPT, body, 0)
    plsc.subcore_barrier()
    pltpu.sync_copy(cnt.at[pl.ds(s * CSTR, CSTR)],
                    out_hbm.at[c, pl.ds(s * CSTR, CSTR)])


def _mlp_body(g_ref, w2_ref, b2_ref, w3_ref, b3_ref, w4_ref, b4_ref, out_ref):
    z = jnp.maximum(g_ref[0], 0.0).astype(jnp.bfloat16)
    z = jnp.maximum(jnp.dot(z, w2_ref[0], preferred_element_type=jnp.float32)
                    + b2_ref[0], 0.0).astype(jnp.bfloat16)
    z = jnp.maximum(jnp.dot(z, w3_ref[0], preferred_element_type=jnp.float32)
                    + b3_ref[0], 0.0).astype(jnp.bfloat16)
    out_ref[0] = (jnp.dot(z, w4_ref[0], preferred_element_type=jnp.float32)
                  + b4_ref[0])


_mlp_call = pl.pallas_call(
    _mlp_body,
    grid=(2, EP // BE),
    in_specs=[
        pl.BlockSpec((1, BE, H), lambda t, i: (t, i, 0)),
        pl.BlockSpec((1, H, H), lambda t, i: (t, 0, 0)),
        pl.BlockSpec((1, 1, H), lambda t, i: (t, 0, 0)),
        pl.BlockSpec((1, H, H), lambda t, i: (t, 0, 0)),
        pl.BlockSpec((1, 1, H), lambda t, i: (t, 0, 0)),
        pl.BlockSpec((1, H, H), lambda t, i: (t, 0, 0)),
        pl.BlockSpec((1, 1, H), lambda t, i: (t, 0, 0)),
    ],
    out_specs=pl.BlockSpec((1, BE, H), lambda t, i: (t, i, 0)),
    out_shape=jax.ShapeDtypeStruct((2, EP, H), jnp.float32),
)


def _pair_mlp_body(emb_ref, w1_ref, b1_ref, w2_ref, b2_ref, w3_ref, b3_ref,
                   w4_ref, b4_ref, out_ref):
    e = emb_ref[...]
    rows = [jnp.concatenate([e[a], e[b]]) for a in range(3) for b in range(3)]
    z0 = jnp.concatenate([jnp.stack(rows),
                          jnp.zeros((7, 2 * H), jnp.float32)]
                         ).astype(jnp.bfloat16)
    for t in range(2):
        z = jnp.maximum(jnp.dot(z0, w1_ref[t], preferred_element_type=jnp.float32)
                        + b1_ref[t], 0.0).astype(jnp.bfloat16)
        z = jnp.maximum(jnp.dot(z, w2_ref[t], preferred_element_type=jnp.float32)
                        + b2_ref[t], 0.0).astype(jnp.bfloat16)
        z = jnp.maximum(jnp.dot(z, w3_ref[t], preferred_element_type=jnp.float32)
                        + b3_ref[t], 0.0).astype(jnp.bfloat16)
        out_ref[t] = (jnp.dot(z, w4_ref[t], preferred_element_type=jnp.float32)
                      + b4_ref[t])


_pair_mlp_call = pl.pallas_call(
    _pair_mlp_body,
    out_shape=jax.ShapeDtypeStruct((2, 16, H), jnp.float32),
)


def _combine_body(cnt_ref, q_ref, m9_ref, out_ref):
    q = q_ref[0, 0]
    qc = q[:, None]
    m9 = m9_ref[...]
    for t in range(2):
        cnt = cnt_ref[t]
        acc = None
        for c in range(3):
            row = jnp.where(qc == 0, m9[t, c * 3 + 0][None, :],
                            jnp.where(qc == 1, m9[t, c * 3 + 1][None, :],
                                      m9[t, c * 3 + 2][None, :]))
            term = cnt[:, c][:, None] * row
            acc = term if acc is None else acc + term
        out_ref[t] = acc


_combine_call = pl.pallas_call(
    _combine_body,
    grid=(NP // BN,),
    in_specs=[
        pl.BlockSpec((2, BN, 8), lambda i: (0, i, 0)),
        pl.BlockSpec((1, 1, BN), lambda i: (i, 0, 0)),
        pl.BlockSpec((2, 16, H), lambda i: (0, 0, 0)),
    ],
    out_specs=pl.BlockSpec((2, BN, H), lambda i: (0, i, 0)),
    out_shape=jax.ShapeDtypeStruct((2, NP, H), jnp.float32),
)


def _init_body(q_ref, emb_ref, wp_ref, bp_ref, x_ref, ab_ref):
    q = q_ref[0, 0]
    e = emb_ref[...]
    qc = q[:, None]
    x = jnp.where(qc == 0, e[0][None, :],
                  jnp.where(qc == 1, e[1][None, :], e[2][None, :]))
    x_ref[...] = x
    xb = x.astype(jnp.bfloat16)
    for t in range(4):
        ab_ref[t] = (jnp.dot(xb, wp_ref[t], preferred_element_type=jnp.float32)
                     + bp_ref[t])


_init_call = pl.pallas_call(
    _init_body,
    grid=(NP // BN,),
    in_specs=[
        pl.BlockSpec((1, 1, BN), lambda i: (i, 0, 0)),
        pl.BlockSpec((3, H), lambda i: (0, 0)),
        pl.BlockSpec((4, H, H), lambda i: (0, 0, 0)),
        pl.BlockSpec((4, 1, H), lambda i: (0, 0, 0)),
    ],
    out_specs=[
        pl.BlockSpec((BN, H), lambda i: (i, 0)),
        pl.BlockSpec((4, BN, H), lambda i: (0, i, 0)),
    ],
    out_shape=[
        jax.ShapeDtypeStruct((NP, H), jnp.float32),
        jax.ShapeDtypeStruct((4, NP, H), jnp.float32),
    ],
)


def _lstm_math(x, mI, mE, h, c, wih, whh):
    gates = (jnp.dot(x, wih[:H], preferred_element_type=jnp.float32)
             + jnp.dot(mI, wih[H:2 * H], preferred_element_type=jnp.float32)
             + jnp.dot(mE, wih[2 * H:], preferred_element_type=jnp.float32)
             + jnp.dot(h, whh, preferred_element_type=jnp.float32))
    i_g = gates[:, :H]
    f_g = gates[:, H:2 * H]
    g_g = gates[:, 2 * H:3 * H]
    o_g = gates[:, 3 * H:]
    c_new = jax.nn.sigmoid(f_g) * c + jax.nn.sigmoid(i_g) * jnp.tanh(g_g)
    h_new = jax.nn.sigmoid(o_g) * jnp.tanh(c_new)
    return h_new, c_new


def _lstm_proj_body(x_ref, aggI_ref, aggE_ref, h_ref, c_ref, wih_ref, whh_ref,
                    wp_ref, bp_ref, h_out, c_out, ab_out):
    h_new, c_new = _lstm_math(x_ref[...], aggI_ref[0], aggE_ref[0],
                              h_ref[...], c_ref[...], wih_ref[...], whh_ref[...])
    h_out[...] = h_new
    c_out[...] = c_new
    hb = h_new.astype(jnp.bfloat16)
    for t in range(4):
        ab_out[t] = (jnp.dot(hb, wp_ref[t], preferred_element_type=jnp.float32)
                     + bp_ref[t])


_lstm_proj_call = pl.pallas_call(
    _lstm_proj_body,
    grid=(NP // BN,),
    in_specs=[
        pl.BlockSpec((BN, H), lambda i: (i, 0)),
        pl.BlockSpec((1, BN, H), lambda i: (0, i, 0)),
        pl.BlockSpec((1, BN, H), lambda i: (1, i, 0)),
        pl.BlockSpec((BN, H), lambda i: (i, 0)),
        pl.BlockSpec((BN, H), lambda i: (i, 0)),
        pl.BlockSpec((3 * H, 4 * H), lambda i: (0, 0)),
        pl.BlockSpec((H, 4 * H), lambda i: (0, 0)),
        pl.BlockSpec((4, H, H), lambda i: (0, 0, 0)),
        pl.BlockSpec((4, 1, H), lambda i: (0, 0, 0)),
    ],
    out_specs=[
        pl.BlockSpec((BN, H), lambda i: (i, 0)),
        pl.BlockSpec((BN, H), lambda i: (i, 0)),
        pl.BlockSpec((4, BN, H), lambda i: (0, i, 0)),
    ],
    out_shape=[
        jax.ShapeDtypeStruct((NP, H), jnp.float32),
        jax.ShapeDtypeStruct((NP, H), jnp.float32),
        jax.ShapeDtypeStruct((4, NP, H), jnp.float32),
    ],
)


def _lstm_score_body(x_ref, aggI_ref, aggE_ref, h_ref, c_ref, wih_ref,
                     whh_ref, ws_ref, out_ref):
    h_new, _ = _lstm_math(x_ref[...], aggI_ref[0], aggE_ref[0],
                          h_ref[...], c_ref[...], wih_ref[...], whh_ref[...])
    out_ref[...] = jnp.sum(h_new * ws_ref[0][None, :], axis=1)[None, :]


_lstm_score_call = pl.pallas_call(
    _lstm_score_body,
    grid=(NP // BN,),
    in_specs=[
        pl.BlockSpec((BN, H), lambda i: (i, 0)),
        pl.BlockSpec((1, BN, H), lambda i: (0, i, 0)),
        pl.BlockSpec((1, BN, H), lambda i: (1, i, 0)),
        pl.BlockSpec((BN, H), lambda i: (i, 0)),
        pl.BlockSpec((BN, H), lambda i: (i, 0)),
        pl.BlockSpec((3 * H, 4 * H), lambda i: (0, 0)),
        pl.BlockSpec((H, 4 * H), lambda i: (0, 0)),
        pl.BlockSpec((1, H), lambda i: (0, 0)),
    ],
    out_specs=pl.BlockSpec((1, BN), lambda i: (0, i)),
    out_shape=jax.ShapeDtypeStruct((1, NP), jnp.float32),
)


def _pad_to(v, length, fill):
    return jnp.concatenate(
        [v, jnp.full((length - v.shape[0],), fill, dtype=v.dtype)])


def kernel(cell_q, edge_intra, edge_inter, params):
    p = params

    wI, wE = p['intra_Ws'], p['inter_Ws']
    bI, bE = p['intra_bs'], p['inter_bs']
    wp = jnp.stack([wI[0][:H], wI[0][H:], wE[0][:H], wE[0][H:]]
                   ).astype(jnp.bfloat16)
    zb = jnp.zeros((H,), jnp.float32)
    bp = jnp.stack([zb, bI[0], zb, bE[0]]).reshape(4, 1, H)
    w2 = jnp.stack([wI[1], wE[1]]).astype(jnp.bfloat16)
    b2 = jnp.stack([bI[1], bE[1]]).reshape(2, 1, H)
    w3 = jnp.stack([wI[2], wE[2]]).astype(jnp.bfloat16)
    b3 = jnp.stack([bI[2], bE[2]]).reshape(2, 1, H)
    w4 = jnp.stack([wI[3], wE[3]]).astype(jnp.bfloat16)
    b4 = jnp.stack([bI[3], bE[3]]).reshape(2, 1, H)

    def gidx_type(edges, a_slab, b_slab):
        src = edges[0].astype(jnp.int32)
        dst = edges[1].astype(jnp.int32)
        ia = _pad_to(src + a_slab * NP, EP, 0)
        ib = _pad_to(dst + b_slab * NP, EP, 0)
        return jnp.stack([ia, ib]).reshape(2, NS, CPT, CH)

    gidx = jnp.stack([gidx_type(edge_intra, 0, 1),
                      gidx_type(edge_inter, 2, 3)])
    sidx = jnp.stack([
        _pad_to(edge_intra[1].astype(jnp.int32), EP, DUMMY),
        _pad_to(edge_inter[1].astype(jnp.int32), EP, DUMMY),
    ]).reshape(2, NS, CPT, CH)

    cq32 = cell_q.astype(jnp.int32)
    ridx = jnp.stack([
        _pad_to(edge_intra[0].astype(jnp.int32), EP, 0),
        _pad_to(edge_inter[0].astype(jnp.int32), EP, 0),
    ]).reshape(2, NS, CPT, CH)

    qpad = _pad_to(cq32, NP, 0)
    q3 = qpad.reshape(NP // BN, 1, BN)
    x, ab = _init_call(q3, p['digit_embed'], wp, bp)

    h = jnp.zeros((NP, H), jnp.float32)
    c = jnp.zeros((NP, H), jnp.float32)
    wih = p['W_ih']
    whh = p['W_hh']
    ws2d = p['w_score'].reshape(1, H)

    w1s = jnp.stack([wI[0], wE[0]]).astype(jnp.bfloat16)
    b1s = jnp.stack([bI[0], bE[0]]).reshape(2, 1, H)
    m9 = _pair_mlp_call(p['digit_embed'], w1s, b1s, w2, b2, w3, b3, w4, b4)
    cnt = _sc_count(qpad, sidx, ridx).reshape(2, NP, 8)
    agg = _combine_call(cnt, q3, m9)
    h, c, ab = _lstm_proj_call(x, agg, agg, h, c, wih, whh, wp, bp)

    for step in range(1, STEPS):
        g = _sc_gather(ab.reshape(4 * NP, H), gidx)
        m = _mlp_call(g, w2, b2, w3, b3, w4, b4)
        agg = _sc_scatter(m, sidx)
        if step < STEPS - 1:
            h, c, ab = _lstm_proj_call(x, agg, agg, h, c, wih, whh, wp, bp)
        else:
            logits = _lstm_score_call(x, agg, agg, h, c, wih, whh, ws2d)

    return logits[0, :N]


# speedup vs baseline: 1.4005x; 1.1142x over previous
import functools

import jax
import jax.numpy as jnp
from jax import lax
from jax.experimental import pallas as pl
from jax.experimental.pallas import tpu as pltpu
from jax.experimental.pallas import tpu_sc as plsc

N = 10000
H = 128
E = 160000
STEPS = 4

NC = 2
NS = 16
CH = 128
CPT = 79
TPT = CPT * CH
EP = NS * TPT
NP = 10240
DUMMY = N

BN = 512
BE = 2048

_mesh = plsc.VectorSubcoreMesh(core_axis_name="c", subcore_axis_name="s")


NBUF = 3


@functools.partial(
    pl.kernel,
    out_type=jax.ShapeDtypeStruct((2, EP, H), jnp.float32),
    mesh=_mesh,
    scratch_types=[
        pltpu.VMEM((CPT, CH), jnp.int32),
        pltpu.VMEM((CPT, CH), jnp.int32),
        pltpu.VMEM((NBUF, CH, H), jnp.float32),
        pltpu.VMEM((NBUF, CH, H), jnp.float32),
        pltpu.SemaphoreType.DMA,
        pltpu.SemaphoreType.DMA,
        pltpu.SemaphoreType.DMA,
    ],
)
def _sc_gather(ab_hbm, gidx_hbm, out_hbm, idxA, idxB, bufA, bufB,
               semA, semB, semO):
    c = lax.axis_index("c")
    s = lax.axis_index("s")
    base = s * TPT

    pltpu.sync_copy(gidx_hbm.at[c, 0, s], idxA)
    pltpu.sync_copy(gidx_hbm.at[c, 1, s], idxB)

    def start_gather(j, slot):
        pltpu.async_copy(ab_hbm.at[idxA.at[j]], bufA.at[slot], semA)
        pltpu.async_copy(ab_hbm.at[idxB.at[j]], bufB.at[slot], semB)

    def wait_gather(slot):
        pltpu.make_async_copy(ab_hbm.at[pl.ds(0, CH)], bufA.at[slot], semA).wait()
        pltpu.make_async_copy(ab_hbm.at[pl.ds(0, CH)], bufB.at[slot], semB).wait()

    def wait_out(slot=0):
        pltpu.make_async_copy(ab_hbm.at[pl.ds(0, CH)], bufA.at[slot], semO).wait()

    for j in range(NBUF - 1):
        start_gather(j, j)

    def body(j, _):
        slot = lax.rem(j, NBUF)
        wait_gather(slot)

        @pl.when(j + NBUF - 1 < CPT)
        def _():
            @pl.when(j >= 1)
            def _():
                wait_out()
            start_gather(j + NBUF - 1, lax.rem(j + NBUF - 1, NBUF))

        @plsc.parallel_loop(0, CH, 1, unroll=8)
        def _add_row(r):
            for k in range(H // 16):
                sl = (slot, r, pl.ds(k * 16, 16))
                bufA[sl] = bufA[sl] + bufB[sl]
        pltpu.async_copy(bufA.at[slot],
                         out_hbm.at[c, pl.ds(base + j * CH, CH)], semO)
        return 0

    lax.fori_loop(0, CPT, body, 0)
    for _ in range(min(NBUF, CPT)):
        wait_out()


@functools.partial(
    pl.kernel,
    out_type=jax.ShapeDtypeStruct((2, NP, H), jnp.float32),
    mesh=_mesh,
    scratch_types=[
        pltpu.VMEM((CPT, CH), jnp.int32),
        pltpu.VMEM((2, CH, H), jnp.float32),
        pltpu.VMEM_SHARED((NP, H), jnp.float32),
        pltpu.SemaphoreType.DMA,
    ],
)
def _sc_scatter(m_hbm, sidx_hbm, out_hbm, idxD, bufM, agg, semM):
    c = lax.axis_index("c")
    s = lax.axis_index("s")
    base = s * TPT
    stripe = NP // NS

    pltpu.sync_copy(sidx_hbm.at[c, s], idxD)

    def zero_row(r, _):
        for k in range(H // 16):
            bufM[0, r, pl.ds(k * 16, 16)] = jnp.zeros((16,), jnp.float32)
        return 0

    lax.fori_loop(0, CH, zero_row, 0)
    for t in range(stripe // CH):
        pltpu.sync_copy(bufM.at[0], agg.at[pl.ds(s * stripe + t * CH, CH)])
    plsc.subcore_barrier()

    def start_load(j, slot):
        pltpu.async_copy(m_hbm.at[c, pl.ds(base + j * CH, CH)],
                         bufM.at[slot], semM)

    def wait_load(slot):
        pltpu.make_async_copy(m_hbm.at[0, pl.ds(0, CH)], bufM.at[slot], semM).wait()

    start_load(0, 0)

    def body(j, _):
        slot = lax.rem(j, 2)
        wait_load(slot)

        @pl.when(j + 1 < CPT)
        def _():
            start_load(j + 1, 1 - slot)

        pltpu.sync_copy(bufM.at[slot], agg.at[idxD.at[j]], add=True)
        return 0

    lax.fori_loop(0, CPT, body, 0)
    plsc.subcore_barrier()
    for t in range(stripe // CH):
        sl = pl.ds(s * stripe + t * CH, CH)
        pltpu.sync_copy(agg.at[sl], out_hbm.at[c, sl])


CSTR = NP * 8 // NS


@functools.partial(
    pl.kernel,
    out_type=jax.ShapeDtypeStruct((2, NP * 8), jnp.float32),
    mesh=_mesh,
    scratch_types=[
        pltpu.VMEM((CPT, CH), jnp.int32),
        pltpu.VMEM((CPT, CH), jnp.int32),
        pltpu.VMEM((2, CH), jnp.int32),
        pltpu.VMEM((1, CH), jnp.int32),
        pltpu.VMEM((CH,), jnp.float32),
        pltpu.VMEM((CSTR,), jnp.float32),
        pltpu.VMEM_SHARED((NP * 8,), jnp.float32),
        pltpu.SemaphoreType.DMA,
    ],
)
def _sc_count(q_hbm, didx_hbm, ridx_hbm, out_hbm,
              idxD, idxS, qbuf, ibuf, ones, zstage, cnt, semQ):
    c = lax.axis_index("c")
    s = lax.axis_index("s")

    pltpu.sync_copy(didx_hbm.at[c, s], idxD)
    pltpu.sync_copy(ridx_hbm.at[c, s], idxS)

    def zrow(r, _):
        zstage[pl.ds(r * 16, 16)] = jnp.zeros((16,), jnp.float32)
        return 0

    lax.fori_loop(0, CSTR // 16, zrow, 0)
    for r in range(CH // 16):
        ones[pl.ds(r * 16, 16)] = jnp.full((16,), 1.0, jnp.float32)
    pltpu.sync_copy(zstage, cnt.at[pl.ds(s * CSTR, CSTR)])
    plsc.subcore_barrier()

    def start_q(j, slot):
        pltpu.async_copy(q_hbm.at[idxS.at[j]], qbuf.at[slot], semQ)

    def wait_q(slot):
        pltpu.make_async_copy(q_hbm.at[pl.ds(0, CH)], qbuf.at[slot], semQ).wait()

    start_q(0, 0)

    def body(j, _):
        slot = lax.rem(j, 2)
        wait_q(slot)

        @pl.when(j + 1 < CPT)
        def _():
            start_q(j + 1, 1 - slot)

        for k in range(CH // 16):
            sl = pl.ds(k * 16, 16)
            ibuf[0, sl] = idxD[j, sl] * 8 + qbuf[slot, sl]
        pltpu.sync_copy(ones, cnt.at[ibuf.at[0]], add=True)
        return 0

    lax.fori_loop(0, CPT, body, 0)
    plsc.subcore_barrier()
    pltpu.sync_copy(cnt.at[pl.ds(s * CSTR, CSTR)],
                    out_hbm.at[c, pl.ds(s * CSTR, CSTR)])


def _mlp_body(g_ref, w2_ref, b2_ref, w3_ref, b3_ref, w4_ref, b4_ref, out_ref):
    z = jnp.maximum(g_ref[0], 0.0).astype(jnp.bfloat16)
    z = jnp.maximum(jnp.dot(z, w2_ref[0], preferred_element_type=jnp.float32)
                    + b2_ref[0], 0.0).astype(jnp.bfloat16)
    z = jnp.maximum(jnp.dot(z, w3_ref[0], preferred_element_type=jnp.float32)
                    + b3_ref[0], 0.0).astype(jnp.bfloat16)
    out_ref[0] = (jnp.dot(z, w4_ref[0], preferred_element_type=jnp.float32)
                  + b4_ref[0])


_mlp_call = pl.pallas_call(
    _mlp_body,
    grid=(2, EP // BE),
    in_specs=[
        pl.BlockSpec((1, BE, H), lambda t, i: (t, i, 0)),
        pl.BlockSpec((1, H, H), lambda t, i: (t, 0, 0)),
        pl.BlockSpec((1, 1, H), lambda t, i: (t, 0, 0)),
        pl.BlockSpec((1, H, H), lambda t, i: (t, 0, 0)),
        pl.BlockSpec((1, 1, H), lambda t, i: (t, 0, 0)),
        pl.BlockSpec((1, H, H), lambda t, i: (t, 0, 0)),
        pl.BlockSpec((1, 1, H), lambda t, i: (t, 0, 0)),
    ],
    out_specs=pl.BlockSpec((1, BE, H), lambda t, i: (t, i, 0)),
    out_shape=jax.ShapeDtypeStruct((2, EP, H), jnp.float32),
)


def _pair_mlp_body(emb_ref, w1_ref, b1_ref, w2_ref, b2_ref, w3_ref, b3_ref,
                   w4_ref, b4_ref, out_ref):
    e = emb_ref[...]
    rows = [jnp.concatenate([e[a], e[b]]) for a in range(3) for b in range(3)]
    z0 = jnp.concatenate([jnp.stack(rows),
                          jnp.zeros((7, 2 * H), jnp.float32)]
                         ).astype(jnp.bfloat16)
    for t in range(2):
        z = jnp.maximum(jnp.dot(z0, w1_ref[t], preferred_element_type=jnp.float32)
                        + b1_ref[t], 0.0).astype(jnp.bfloat16)
        z = jnp.maximum(jnp.dot(z, w2_ref[t], preferred_element_type=jnp.float32)
                        + b2_ref[t], 0.0).astype(jnp.bfloat16)
        z = jnp.maximum(jnp.dot(z, w3_ref[t], preferred_element_type=jnp.float32)
                        + b3_ref[t], 0.0).astype(jnp.bfloat16)
        out_ref[t] = (jnp.dot(z, w4_ref[t], preferred_element_type=jnp.float32)
                      + b4_ref[t])


_pair_mlp_call = pl.pallas_call(
    _pair_mlp_body,
    out_shape=jax.ShapeDtypeStruct((2, 16, H), jnp.float32),
)


def _combine_body(cnt_ref, q_ref, m9_ref, out_ref):
    q = q_ref[0, 0]
    qc = q[:, None]
    m9 = m9_ref[...]
    for t in range(2):
        cnt = cnt_ref[t]
        acc = None
        for c in range(3):
            row = jnp.where(qc == 0, m9[t, c * 3 + 0][None, :],
                            jnp.where(qc == 1, m9[t, c * 3 + 1][None, :],
                                      m9[t, c * 3 + 2][None, :]))
            term = cnt[:, c][:, None] * row
            acc = term if acc is None else acc + term
        out_ref[t] = acc


_combine_call = pl.pallas_call(
    _combine_body,
    grid=(NP // BN,),
    in_specs=[
        pl.BlockSpec((2, BN, 8), lambda i: (0, i, 0)),
        pl.BlockSpec((1, 1, BN), lambda i: (i, 0, 0)),
        pl.BlockSpec((2, 16, H), lambda i: (0, 0, 0)),
    ],
    out_specs=pl.BlockSpec((2, BN, H), lambda i: (0, i, 0)),
    out_shape=jax.ShapeDtypeStruct((2, NP, H), jnp.float32),
)


def _init_body(q_ref, emb_ref, wp_ref, bp_ref, x_ref, ab_ref):
    q = q_ref[0, 0]
    e = emb_ref[...]
    qc = q[:, None]
    x = jnp.where(qc == 0, e[0][None, :],
                  jnp.where(qc == 1, e[1][None, :], e[2][None, :]))
    x_ref[...] = x
    xb = x.astype(jnp.bfloat16)
    for t in range(4):
        ab_ref[t] = (jnp.dot(xb, wp_ref[t], preferred_element_type=jnp.float32)
                     + bp_ref[t])


_init_call = pl.pallas_call(
    _init_body,
    grid=(NP // BN,),
    in_specs=[
        pl.BlockSpec((1, 1, BN), lambda i: (i, 0, 0)),
        pl.BlockSpec((3, H), lambda i: (0, 0)),
        pl.BlockSpec((4, H, H), lambda i: (0, 0, 0)),
        pl.BlockSpec((4, 1, H), lambda i: (0, 0, 0)),
    ],
    out_specs=[
        pl.BlockSpec((BN, H), lambda i: (i, 0)),
        pl.BlockSpec((4, BN, H), lambda i: (0, i, 0)),
    ],
    out_shape=[
        jax.ShapeDtypeStruct((NP, H), jnp.float32),
        jax.ShapeDtypeStruct((4, NP, H), jnp.float32),
    ],
)


def _lstm_math(x, mI, mE, h, c, wih, whh):
    gates = (jnp.dot(x, wih[:H], preferred_element_type=jnp.float32)
             + jnp.dot(mI, wih[H:2 * H], preferred_element_type=jnp.float32)
             + jnp.dot(mE, wih[2 * H:], preferred_element_type=jnp.float32)
             + jnp.dot(h, whh, preferred_element_type=jnp.float32))
    i_g = gates[:, :H]
    f_g = gates[:, H:2 * H]
    g_g = gates[:, 2 * H:3 * H]
    o_g = gates[:, 3 * H:]
    c_new = jax.nn.sigmoid(f_g) * c + jax.nn.sigmoid(i_g) * jnp.tanh(g_g)
    h_new = jax.nn.sigmoid(o_g) * jnp.tanh(c_new)
    return h_new, c_new


def _lstm_proj_body(x_ref, aggI_ref, aggE_ref, h_ref, c_ref, wih_ref, whh_ref,
                    wp_ref, bp_ref, h_out, c_out, ab_out):
    h_new, c_new = _lstm_math(x_ref[...], aggI_ref[0], aggE_ref[0],
                              h_ref[...], c_ref[...], wih_ref[...], whh_ref[...])
    h_out[...] = h_new
    c_out[...] = c_new
    hb = h_new.astype(jnp.bfloat16)
    for t in range(4):
        ab_out[t] = (jnp.dot(hb, wp_ref[t], preferred_element_type=jnp.float32)
                     + bp_ref[t])


_lstm_proj_call = pl.pallas_call(
    _lstm_proj_body,
    grid=(NP // BN,),
    in_specs=[
        pl.BlockSpec((BN, H), lambda i: (i, 0)),
        pl.BlockSpec((1, BN, H), lambda i: (0, i, 0)),
        pl.BlockSpec((1, BN, H), lambda i: (1, i, 0)),
        pl.BlockSpec((BN, H), lambda i: (i, 0)),
        pl.BlockSpec((BN, H), lambda i: (i, 0)),
        pl.BlockSpec((3 * H, 4 * H), lambda i: (0, 0)),
        pl.BlockSpec((H, 4 * H), lambda i: (0, 0)),
        pl.BlockSpec((4, H, H), lambda i: (0, 0, 0)),
        pl.BlockSpec((4, 1, H), lambda i: (0, 0, 0)),
    ],
    out_specs=[
        pl.BlockSpec((BN, H), lambda i: (i, 0)),
        pl.BlockSpec((BN, H), lambda i: (i, 0)),
        pl.BlockSpec((4, BN, H), lambda i: (0, i, 0)),
    ],
    out_shape=[
        jax.ShapeDtypeStruct((NP, H), jnp.float32),
        jax.ShapeDtypeStruct((NP, H), jnp.float32),
        jax.ShapeDtypeStruct((4, NP, H), jnp.float32),
    ],
)


def _lstm_score_body(x_ref, aggI_ref, aggE_ref, h_ref, c_ref, wih_ref,
                     whh_ref, ws_ref, out_ref):
    h_new, _ = _lstm_math(x_ref[...], aggI_ref[0], aggE_ref[0],
                          h_ref[...], c_ref[...], wih_ref[...], whh_ref[...])
    out_ref[...] = jnp.sum(h_new * ws_ref[0][None, :], axis=1)[None, :]


_lstm_score_call = pl.pallas_call(
    _lstm_score_body,
    grid=(NP // BN,),
    in_specs=[
        pl.BlockSpec((BN, H), lambda i: (i, 0)),
        pl.BlockSpec((1, BN, H), lambda i: (0, i, 0)),
        pl.BlockSpec((1, BN, H), lambda i: (1, i, 0)),
        pl.BlockSpec((BN, H), lambda i: (i, 0)),
        pl.BlockSpec((BN, H), lambda i: (i, 0)),
        pl.BlockSpec((3 * H, 4 * H), lambda i: (0, 0)),
        pl.BlockSpec((H, 4 * H), lambda i: (0, 0)),
        pl.BlockSpec((1, H), lambda i: (0, 0)),
    ],
    out_specs=pl.BlockSpec((1, BN), lambda i: (0, i)),
    out_shape=jax.ShapeDtypeStruct((1, NP), jnp.float32),
)


def _pad_to(v, length, fill):
    return jnp.concatenate(
        [v, jnp.full((length - v.shape[0],), fill, dtype=v.dtype)])


def kernel(cell_q, edge_intra, edge_inter, params):
    p = params

    wI, wE = p['intra_Ws'], p['inter_Ws']
    bI, bE = p['intra_bs'], p['inter_bs']
    wp = jnp.stack([wI[0][:H], wI[0][H:], wE[0][:H], wE[0][H:]]
                   ).astype(jnp.bfloat16)
    zb = jnp.zeros((H,), jnp.float32)
    bp = jnp.stack([zb, bI[0], zb, bE[0]]).reshape(4, 1, H)
    w2 = jnp.stack([wI[1], wE[1]]).astype(jnp.bfloat16)
    b2 = jnp.stack([bI[1], bE[1]]).reshape(2, 1, H)
    w3 = jnp.stack([wI[2], wE[2]]).astype(jnp.bfloat16)
    b3 = jnp.stack([bI[2], bE[2]]).reshape(2, 1, H)
    w4 = jnp.stack([wI[3], wE[3]]).astype(jnp.bfloat16)
    b4 = jnp.stack([bI[3], bE[3]]).reshape(2, 1, H)

    def gidx_type(edges, a_slab, b_slab):
        src = edges[0].astype(jnp.int32)
        dst = edges[1].astype(jnp.int32)
        ia = _pad_to(src + a_slab * NP, EP, 0)
        ib = _pad_to(dst + b_slab * NP, EP, 0)
        return jnp.stack([ia, ib]).reshape(2, NS, CPT, CH)

    gidx = jnp.stack([gidx_type(edge_intra, 0, 1),
                      gidx_type(edge_inter, 2, 3)])
    sidx = jnp.stack([
        _pad_to(edge_intra[1].astype(jnp.int32), EP, DUMMY),
        _pad_to(edge_inter[1].astype(jnp.int32), EP, DUMMY),
    ]).reshape(2, NS, CPT, CH)

    cq32 = cell_q.astype(jnp.int32)
    ridx = jnp.stack([
        _pad_to(edge_intra[0].astype(jnp.int32), EP, 0),
        _pad_to(edge_inter[0].astype(jnp.int32), EP, 0),
    ]).reshape(2, NS, CPT, CH)

    qpad = _pad_to(cq32, NP, 0)
    q3 = qpad.reshape(NP // BN, 1, BN)
    x, ab = _init_call(q3, p['digit_embed'], wp, bp)

    h = jnp.zeros((NP, H), jnp.float32)
    c = jnp.zeros((NP, H), jnp.float32)
    wih = p['W_ih']
    whh = p['W_hh']
    ws2d = p['w_score'].reshape(1, H)

    w1s = jnp.stack([wI[0], wE[0]]).astype(jnp.bfloat16)
    b1s = jnp.stack([bI[0], bE[0]]).reshape(2, 1, H)
    m9 = _pair_mlp_call(p['digit_embed'], w1s, b1s, w2, b2, w3, b3, w4, b4)
    cnt = _sc_count(qpad, sidx, ridx).reshape(2, NP, 8)
    agg = _combine_call(cnt, q3, m9)
    h, c, ab = _lstm_proj_call(x, agg, agg, h, c, wih, whh, wp, bp)

    for step in range(1, STEPS):
        g = _sc_gather(ab.reshape(4 * NP, H), gidx)
        m = _mlp_call(g, w2, b2, w3, b3, w4, b4)
        agg = _sc_scatter(m, sidx)
        if step < STEPS - 1:
            h, c, ab = _lstm_proj_call(x, agg, agg, h, c, wih, whh, wp, bp)
        else:
            logits = _lstm_score_call(x, agg, agg, h, c, wih, whh, ws2d)

    return logits[0, :N]


# speedup vs baseline: 1.4122x; 1.0084x over previous
import functools

import jax
import jax.numpy as jnp
from jax import lax
from jax.experimental import pallas as pl
from jax.experimental.pallas import tpu as pltpu
from jax.experimental.pallas import tpu_sc as plsc

N = 10000
H = 128
E = 160000
STEPS = 4

NC = 2
NS = 16
CH = 128
CPT = 79
TPT = CPT * CH
EP = NS * TPT
NP = 10240
DUMMY = N

BN = 1024
BE = 2048

_mesh = plsc.VectorSubcoreMesh(core_axis_name="c", subcore_axis_name="s")


NBUF = 3


@functools.partial(
    pl.kernel,
    out_type=jax.ShapeDtypeStruct((2, EP, H), jnp.float32),
    mesh=_mesh,
    scratch_types=[
        pltpu.VMEM((CPT, CH), jnp.int32),
        pltpu.VMEM((CPT, CH), jnp.int32),
        pltpu.VMEM((NBUF, CH, H), jnp.float32),
        pltpu.VMEM((NBUF, CH, H), jnp.float32),
        pltpu.SemaphoreType.DMA,
        pltpu.SemaphoreType.DMA,
        pltpu.SemaphoreType.DMA,
    ],
)
def _sc_gather(ab_hbm, gidx_hbm, out_hbm, idxA, idxB, bufA, bufB,
               semA, semB, semO):
    c = lax.axis_index("c")
    s = lax.axis_index("s")
    base = s * TPT

    pltpu.sync_copy(gidx_hbm.at[c, 0, s], idxA)
    pltpu.sync_copy(gidx_hbm.at[c, 1, s], idxB)

    def start_gather(j, slot):
        pltpu.async_copy(ab_hbm.at[idxA.at[j]], bufA.at[slot], semA)
        pltpu.async_copy(ab_hbm.at[idxB.at[j]], bufB.at[slot], semB)

    def wait_gather(slot):
        pltpu.make_async_copy(ab_hbm.at[pl.ds(0, CH)], bufA.at[slot], semA).wait()
        pltpu.make_async_copy(ab_hbm.at[pl.ds(0, CH)], bufB.at[slot], semB).wait()

    def wait_out(slot=0):
        pltpu.make_async_copy(ab_hbm.at[pl.ds(0, CH)], bufA.at[slot], semO).wait()

    for j in range(NBUF - 1):
        start_gather(j, j)

    def body(j, _):
        slot = lax.rem(j, NBUF)
        wait_gather(slot)

        @pl.when(j + NBUF - 1 < CPT)
        def _():
            @pl.when(j >= 1)
            def _():
                wait_out()
            start_gather(j + NBUF - 1, lax.rem(j + NBUF - 1, NBUF))

        @plsc.parallel_loop(0, CH, 1, unroll=8)
        def _add_row(r):
            for k in range(H // 16):
                sl = (slot, r, pl.ds(k * 16, 16))
                bufA[sl] = bufA[sl] + bufB[sl]
        pltpu.async_copy(bufA.at[slot],
                         out_hbm.at[c, pl.ds(base + j * CH, CH)], semO)
        return 0

    lax.fori_loop(0, CPT, body, 0)
    for _ in range(min(NBUF, CPT)):
        wait_out()


@functools.partial(
    pl.kernel,
    out_type=jax.ShapeDtypeStruct((2, NP, H), jnp.float32),
    mesh=_mesh,
    scratch_types=[
        pltpu.VMEM((CPT, CH), jnp.int32),
        pltpu.VMEM((2, CH, H), jnp.float32),
        pltpu.VMEM_SHARED((NP, H), jnp.float32),
        pltpu.SemaphoreType.DMA,
    ],
)
def _sc_scatter(m_hbm, sidx_hbm, out_hbm, idxD, bufM, agg, semM):
    c = lax.axis_index("c")
    s = lax.axis_index("s")
    base = s * TPT
    stripe = NP // NS

    pltpu.sync_copy(sidx_hbm.at[c, s], idxD)

    def zero_row(r, _):
        for k in range(H // 16):
            bufM[0, r, pl.ds(k * 16, 16)] = jnp.zeros((16,), jnp.float32)
        return 0

    lax.fori_loop(0, CH, zero_row, 0)
    for t in range(stripe // CH):
        pltpu.sync_copy(bufM.at[0], agg.at[pl.ds(s * stripe + t * CH, CH)])
    plsc.subcore_barrier()

    def start_load(j, slot):
        pltpu.async_copy(m_hbm.at[c, pl.ds(base + j * CH, CH)],
                         bufM.at[slot], semM)

    def wait_load(slot):
        pltpu.make_async_copy(m_hbm.at[0, pl.ds(0, CH)], bufM.at[slot], semM).wait()

    start_load(0, 0)

    def body(j, _):
        slot = lax.rem(j, 2)
        wait_load(slot)

        @pl.when(j + 1 < CPT)
        def _():
            start_load(j + 1, 1 - slot)

        pltpu.sync_copy(bufM.at[slot], agg.at[idxD.at[j]], add=True)
        return 0

    lax.fori_loop(0, CPT, body, 0)
    plsc.subcore_barrier()
    for t in range(stripe // CH):
        sl = pl.ds(s * stripe + t * CH, CH)
        pltpu.sync_copy(agg.at[sl], out_hbm.at[c, sl])


CSTR = NP * 8 // NS


@functools.partial(
    pl.kernel,
    out_type=jax.ShapeDtypeStruct((2, NP * 8), jnp.float32),
    mesh=_mesh,
    scratch_types=[
        pltpu.VMEM((CPT, CH), jnp.int32),
        pltpu.VMEM((CPT, CH), jnp.int32),
        pltpu.VMEM((2, CH), jnp.int32),
        pltpu.VMEM((1, CH), jnp.int32),
        pltpu.VMEM((CH,), jnp.float32),
        pltpu.VMEM((CSTR,), jnp.float32),
        pltpu.VMEM_SHARED((NP * 8,), jnp.float32),
        pltpu.SemaphoreType.DMA,
    ],
)
def _sc_count(q_hbm, didx_hbm, ridx_hbm, out_hbm,
              idxD, idxS, qbuf, ibuf, ones, zstage, cnt, semQ):
    c = lax.axis_index("c")
    s = lax.axis_index("s")

    pltpu.sync_copy(didx_hbm.at[c, s], idxD)
    pltpu.sync_copy(ridx_hbm.at[c, s], idxS)

    def zrow(r, _):
        zstage[pl.ds(r * 16, 16)] = jnp.zeros((16,), jnp.float32)
        return 0

    lax.fori_loop(0, CSTR // 16, zrow, 0)
    for r in range(CH // 16):
        ones[pl.ds(r * 16, 16)] = jnp.full((16,), 1.0, jnp.float32)
    pltpu.sync_copy(zstage, cnt.at[pl.ds(s * CSTR, CSTR)])
    plsc.subcore_barrier()

    def start_q(j, slot):
        pltpu.async_copy(q_hbm.at[idxS.at[j]], qbuf.at[slot], semQ)

    def wait_q(slot):
        pltpu.make_async_copy(q_hbm.at[pl.ds(0, CH)], qbuf.at[slot], semQ).wait()

    start_q(0, 0)

    def body(j, _):
        slot = lax.rem(j, 2)
        wait_q(slot)

        @pl.when(j + 1 < CPT)
        def _():
            start_q(j + 1, 1 - slot)

        for k in range(CH // 16):
            sl = pl.ds(k * 16, 16)
            ibuf[0, sl] = idxD[j, sl] * 8 + qbuf[slot, sl]
        pltpu.sync_copy(ones, cnt.at[ibuf.at[0]], add=True)
        return 0

    lax.fori_loop(0, CPT, body, 0)
    plsc.subcore_barrier()
    pltpu.sync_copy(cnt.at[pl.ds(s * CSTR, CSTR)],
                    out_hbm.at[c, pl.ds(s * CSTR, CSTR)])


def _mlp_body(g_ref, w2_ref, b2_ref, w3_ref, b3_ref, w4_ref, b4_ref, out_ref):
    z = jnp.maximum(g_ref[0], 0.0).astype(jnp.bfloat16)
    z = jnp.maximum(jnp.dot(z, w2_ref[0], preferred_element_type=jnp.float32)
                    + b2_ref[0], 0.0).astype(jnp.bfloat16)
    z = jnp.maximum(jnp.dot(z, w3_ref[0], preferred_element_type=jnp.float32)
                    + b3_ref[0], 0.0).astype(jnp.bfloat16)
    out_ref[0] = (jnp.dot(z, w4_ref[0], preferred_element_type=jnp.float32)
                  + b4_ref[0])


_mlp_call = pl.pallas_call(
    _mlp_body,
    grid=(2, EP // BE),
    in_specs=[
        pl.BlockSpec((1, BE, H), lambda t, i: (t, i, 0)),
        pl.BlockSpec((1, H, H), lambda t, i: (t, 0, 0)),
        pl.BlockSpec((1, 1, H), lambda t, i: (t, 0, 0)),
        pl.BlockSpec((1, H, H), lambda t, i: (t, 0, 0)),
        pl.BlockSpec((1, 1, H), lambda t, i: (t, 0, 0)),
        pl.BlockSpec((1, H, H), lambda t, i: (t, 0, 0)),
        pl.BlockSpec((1, 1, H), lambda t, i: (t, 0, 0)),
    ],
    out_specs=pl.BlockSpec((1, BE, H), lambda t, i: (t, i, 0)),
    out_shape=jax.ShapeDtypeStruct((2, EP, H), jnp.float32),
)


def _pair_mlp_body(emb_ref, w1_ref, b1_ref, w2_ref, b2_ref, w3_ref, b3_ref,
                   w4_ref, b4_ref, out_ref):
    e = emb_ref[...]
    rows = [jnp.concatenate([e[a], e[b]]) for a in range(3) for b in range(3)]
    z0 = jnp.concatenate([jnp.stack(rows),
                          jnp.zeros((7, 2 * H), jnp.float32)]
                         ).astype(jnp.bfloat16)
    for t in range(2):
        z = jnp.maximum(jnp.dot(z0, w1_ref[t], preferred_element_type=jnp.float32)
                        + b1_ref[t], 0.0).astype(jnp.bfloat16)
        z = jnp.maximum(jnp.dot(z, w2_ref[t], preferred_element_type=jnp.float32)
                        + b2_ref[t], 0.0).astype(jnp.bfloat16)
        z = jnp.maximum(jnp.dot(z, w3_ref[t], preferred_element_type=jnp.float32)
                        + b3_ref[t], 0.0).astype(jnp.bfloat16)
        out_ref[t] = (jnp.dot(z, w4_ref[t], preferred_element_type=jnp.float32)
                      + b4_ref[t])


_pair_mlp_call = pl.pallas_call(
    _pair_mlp_body,
    out_shape=jax.ShapeDtypeStruct((2, 16, H), jnp.float32),
)


def _combine_body(cnt_ref, q_ref, m9_ref, out_ref):
    q = q_ref[0, 0]
    qc = q[:, None]
    m9 = m9_ref[...]
    for t in range(2):
        cnt = cnt_ref[t]
        acc = None
        for c in range(3):
            row = jnp.where(qc == 0, m9[t, c * 3 + 0][None, :],
                            jnp.where(qc == 1, m9[t, c * 3 + 1][None, :],
                                      m9[t, c * 3 + 2][None, :]))
            term = cnt[:, c][:, None] * row
            acc = term if acc is None else acc + term
        out_ref[t] = acc


_combine_call = pl.pallas_call(
    _combine_body,
    grid=(NP // BN,),
    in_specs=[
        pl.BlockSpec((2, BN, 8), lambda i: (0, i, 0)),
        pl.BlockSpec((1, 1, BN), lambda i: (i, 0, 0)),
        pl.BlockSpec((2, 16, H), lambda i: (0, 0, 0)),
    ],
    out_specs=pl.BlockSpec((2, BN, H), lambda i: (0, i, 0)),
    out_shape=jax.ShapeDtypeStruct((2, NP, H), jnp.float32),
)


def _init_body(q_ref, emb_ref, wp_ref, bp_ref, x_ref, ab_ref):
    q = q_ref[0, 0]
    e = emb_ref[...]
    qc = q[:, None]
    x = jnp.where(qc == 0, e[0][None, :],
                  jnp.where(qc == 1, e[1][None, :], e[2][None, :]))
    x_ref[...] = x
    xb = x.astype(jnp.bfloat16)
    for t in range(4):
        ab_ref[t] = (jnp.dot(xb, wp_ref[t], preferred_element_type=jnp.float32)
                     + bp_ref[t])


_init_call = pl.pallas_call(
    _init_body,
    grid=(NP // BN,),
    in_specs=[
        pl.BlockSpec((1, 1, BN), lambda i: (i, 0, 0)),
        pl.BlockSpec((3, H), lambda i: (0, 0)),
        pl.BlockSpec((4, H, H), lambda i: (0, 0, 0)),
        pl.BlockSpec((4, 1, H), lambda i: (0, 0, 0)),
    ],
    out_specs=[
        pl.BlockSpec((BN, H), lambda i: (i, 0)),
        pl.BlockSpec((4, BN, H), lambda i: (0, i, 0)),
    ],
    out_shape=[
        jax.ShapeDtypeStruct((NP, H), jnp.float32),
        jax.ShapeDtypeStruct((4, NP, H), jnp.float32),
    ],
)


def _lstm_math(x, mI, mE, h, c, wih, whh):
    gates = (jnp.dot(x, wih[:H], preferred_element_type=jnp.float32)
             + jnp.dot(mI, wih[H:2 * H], preferred_element_type=jnp.float32)
             + jnp.dot(mE, wih[2 * H:], preferred_element_type=jnp.float32)
             + jnp.dot(h, whh, preferred_element_type=jnp.float32))
    i_g = gates[:, :H]
    f_g = gates[:, H:2 * H]
    g_g = gates[:, 2 * H:3 * H]
    o_g = gates[:, 3 * H:]
    c_new = jax.nn.sigmoid(f_g) * c + jax.nn.sigmoid(i_g) * jnp.tanh(g_g)
    h_new = jax.nn.sigmoid(o_g) * jnp.tanh(c_new)
    return h_new, c_new


def _lstm_proj_body(x_ref, aggI_ref, aggE_ref, h_ref, c_ref, wih_ref, whh_ref,
                    wp_ref, bp_ref, h_out, c_out, ab_out):
    h_new, c_new = _lstm_math(x_ref[...], aggI_ref[0], aggE_ref[0],
                              h_ref[...], c_ref[...], wih_ref[...], whh_ref[...])
    h_out[...] = h_new
    c_out[...] = c_new
    hb = h_new.astype(jnp.bfloat16)
    for t in range(4):
        ab_out[t] = (jnp.dot(hb, wp_ref[t], preferred_element_type=jnp.float32)
                     + bp_ref[t])


_lstm_proj_call = pl.pallas_call(
    _lstm_proj_body,
    grid=(NP // BN,),
    in_specs=[
        pl.BlockSpec((BN, H), lambda i: (i, 0)),
        pl.BlockSpec((1, BN, H), lambda i: (0, i, 0)),
        pl.BlockSpec((1, BN, H), lambda i: (1, i, 0)),
        pl.BlockSpec((BN, H), lambda i: (i, 0)),
        pl.BlockSpec((BN, H), lambda i: (i, 0)),
        pl.BlockSpec((3 * H, 4 * H), lambda i: (0, 0)),
        pl.BlockSpec((H, 4 * H), lambda i: (0, 0)),
        pl.BlockSpec((4, H, H), lambda i: (0, 0, 0)),
        pl.BlockSpec((4, 1, H), lambda i: (0, 0, 0)),
    ],
    out_specs=[
        pl.BlockSpec((BN, H), lambda i: (i, 0)),
        pl.BlockSpec((BN, H), lambda i: (i, 0)),
        pl.BlockSpec((4, BN, H), lambda i: (0, i, 0)),
    ],
    out_shape=[
        jax.ShapeDtypeStruct((NP, H), jnp.float32),
        jax.ShapeDtypeStruct((NP, H), jnp.float32),
        jax.ShapeDtypeStruct((4, NP, H), jnp.float32),
    ],
)


def _lstm_score_body(x_ref, aggI_ref, aggE_ref, h_ref, c_ref, wih_ref,
                     whh_ref, ws_ref, out_ref):
    h_new, _ = _lstm_math(x_ref[...], aggI_ref[0], aggE_ref[0],
                          h_ref[...], c_ref[...], wih_ref[...], whh_ref[...])
    out_ref[...] = jnp.sum(h_new * ws_ref[0][None, :], axis=1)[None, :]


_lstm_score_call = pl.pallas_call(
    _lstm_score_body,
    grid=(NP // BN,),
    in_specs=[
        pl.BlockSpec((BN, H), lambda i: (i, 0)),
        pl.BlockSpec((1, BN, H), lambda i: (0, i, 0)),
        pl.BlockSpec((1, BN, H), lambda i: (1, i, 0)),
        pl.BlockSpec((BN, H), lambda i: (i, 0)),
        pl.BlockSpec((BN, H), lambda i: (i, 0)),
        pl.BlockSpec((3 * H, 4 * H), lambda i: (0, 0)),
        pl.BlockSpec((H, 4 * H), lambda i: (0, 0)),
        pl.BlockSpec((1, H), lambda i: (0, 0)),
    ],
    out_specs=pl.BlockSpec((1, BN), lambda i: (0, i)),
    out_shape=jax.ShapeDtypeStruct((1, NP), jnp.float32),
)


def _pad_to(v, length, fill):
    return jnp.concatenate(
        [v, jnp.full((length - v.shape[0],), fill, dtype=v.dtype)])


def kernel(cell_q, edge_intra, edge_inter, params):
    p = params

    wI, wE = p['intra_Ws'], p['inter_Ws']
    bI, bE = p['intra_bs'], p['inter_bs']
    wp = jnp.stack([wI[0][:H], wI[0][H:], wE[0][:H], wE[0][H:]]
                   ).astype(jnp.bfloat16)
    zb = jnp.zeros((H,), jnp.float32)
    bp = jnp.stack([zb, bI[0], zb, bE[0]]).reshape(4, 1, H)
    w2 = jnp.stack([wI[1], wE[1]]).astype(jnp.bfloat16)
    b2 = jnp.stack([bI[1], bE[1]]).reshape(2, 1, H)
    w3 = jnp.stack([wI[2], wE[2]]).astype(jnp.bfloat16)
    b3 = jnp.stack([bI[2], bE[2]]).reshape(2, 1, H)
    w4 = jnp.stack([wI[3], wE[3]]).astype(jnp.bfloat16)
    b4 = jnp.stack([bI[3], bE[3]]).reshape(2, 1, H)

    def gidx_type(edges, a_slab, b_slab):
        src = edges[0].astype(jnp.int32)
        dst = edges[1].astype(jnp.int32)
        ia = _pad_to(src + a_slab * NP, EP, 0)
        ib = _pad_to(dst + b_slab * NP, EP, 0)
        return jnp.stack([ia, ib]).reshape(2, NS, CPT, CH)

    gidx = jnp.stack([gidx_type(edge_intra, 0, 1),
                      gidx_type(edge_inter, 2, 3)])
    sidx = jnp.stack([
        _pad_to(edge_intra[1].astype(jnp.int32), EP, DUMMY),
        _pad_to(edge_inter[1].astype(jnp.int32), EP, DUMMY),
    ]).reshape(2, NS, CPT, CH)

    cq32 = cell_q.astype(jnp.int32)
    ridx = jnp.stack([
        _pad_to(edge_intra[0].astype(jnp.int32), EP, 0),
        _pad_to(edge_inter[0].astype(jnp.int32), EP, 0),
    ]).reshape(2, NS, CPT, CH)

    qpad = _pad_to(cq32, NP, 0)
    q3 = qpad.reshape(NP // BN, 1, BN)
    x, ab = _init_call(q3, p['digit_embed'], wp, bp)

    h = jnp.zeros((NP, H), jnp.float32)
    c = jnp.zeros((NP, H), jnp.float32)
    wih = p['W_ih']
    whh = p['W_hh']
    ws2d = p['w_score'].reshape(1, H)

    w1s = jnp.stack([wI[0], wE[0]]).astype(jnp.bfloat16)
    b1s = jnp.stack([bI[0], bE[0]]).reshape(2, 1, H)
    m9 = _pair_mlp_call(p['digit_embed'], w1s, b1s, w2, b2, w3, b3, w4, b4)
    cnt = _sc_count(qpad, sidx, ridx).reshape(2, NP, 8)
    agg = _combine_call(cnt, q3, m9)
    h, c, ab = _lstm_proj_call(x, agg, agg, h, c, wih, whh, wp, bp)

    for step in range(1, STEPS):
        g = _sc_gather(ab.reshape(4 * NP, H), gidx)
        m = _mlp_call(g, w2, b2, w3, b3, w4, b4)
        agg = _sc_scatter(m, sidx)
        if step < STEPS - 1:
            h, c, ab = _lstm_proj_call(x, agg, agg, h, c, wih, whh, wp, bp)
        else:
            logits = _lstm_score_call(x, agg, agg, h, c, wih, whh, ws2d)

    return logits[0, :N]
